# baseline (speedup 1.0000x reference)
import functools

import jax
import jax.numpy as jnp
from jax import lax
from jax.experimental import pallas as pl
from jax.experimental.pallas import tpu as pltpu
from jax.experimental.pallas import tpu_sc as plsc

N_NODES = 50000
N_EDGES = 800000
LATENT = 64
NODE_TYPE_SIZE = 9
MP_STEPS = 2

_NBLK = 2000
_EBLK = 4096

_NC = 2
_NS = 16
_NW = _NC * _NS
_CH = 128
_NBUF = 4
_EPAD = 802816
_WCH = _EPAD // (_NW * _CH)


def _make_gather(depth, dtype):
    mesh = plsc.VectorSubcoreMesh(core_axis_name="c", subcore_axis_name="s")
    oshape = jax.ShapeDtypeStruct((_NW, _WCH, _CH, depth), dtype)

    @functools.partial(
        pl.kernel, mesh=mesh,
        out_type=(oshape, oshape),
        compiler_params=pltpu.CompilerParams(use_tc_tiling_on_sc=False),
        scratch_types=[
            pltpu.VMEM((_WCH, _CH), jnp.int32),
            pltpu.VMEM((_CH, depth), dtype),
            pltpu.VMEM((_CH, depth), dtype),
            pltpu.VMEM((_CH, depth), dtype),
            pltpu.VMEM((_CH, depth), dtype),
            pltpu.SemaphoreType.DMA,
            pltpu.SemaphoreType.DMA,
        ],
    )
    def gather_k(table, idx_a, idx_b, out_a, out_b, idx_v, r0, r1, r2, r3,
                 gsem, ssem):
        wid = lax.axis_index("s") * _NC + lax.axis_index("c")
        rows = (r0, r1, r2, r3)

        def one_pass(idx_hbm, out_hbm):
            pltpu.sync_copy(idx_hbm.at[wid], idx_v)

            def group(g, _):
                cps = []
                for b in range(_NBUF):
                    j = g * _NBUF + b
                    cps.append(pltpu.async_copy(
                        table.at[idx_v.at[j]], rows[b], gsem))
                sts = []
                for b in range(_NBUF):
                    j = g * _NBUF + b
                    cps[b].wait()
                    sts.append(pltpu.async_copy(
                        rows[b], out_hbm.at[wid, j], ssem))
                for b in range(_NBUF):
                    sts[b].wait()
                return 0

            lax.fori_loop(0, _WCH // _NBUF, group, 0)

        one_pass(idx_a, out_a)
        one_pass(idx_b, out_b)

    return gather_k


_gather64 = _make_gather(LATENT, jnp.bfloat16)
_gather8 = _make_gather(8, jnp.float32)


def _sc_gather_pair(table, idx_a, idx_b, depth):
    k = _gather64 if depth == LATENT else _gather8
    oa, ob = k(table, idx_a, idx_b)
    return (oa.reshape(_EPAD, depth), ob.reshape(_EPAD, depth))


_SCH = _EPAD // (_NS * _CH)
_HALF = N_NODES // 2
_TROWS = 25088
_ZROWS = 112
_ZCH = 14
_CPT = 1568
_SNBUF = 2


def _make_scatter():
    mesh = plsc.VectorSubcoreMesh(core_axis_name="c", subcore_axis_name="s")

    @functools.partial(
        pl.kernel, mesh=mesh,
        out_type=jax.ShapeDtypeStruct((N_NODES, LATENT), jnp.float32),
        compiler_params=pltpu.CompilerParams(use_tc_tiling_on_sc=False),
        scratch_types=[
            pltpu.VMEM_SHARED((_TROWS, LATENT), jnp.float32),
            pltpu.VMEM((_CH,), jnp.int32),
            pltpu.VMEM((_CH,), jnp.int32),
            pltpu.VMEM((_CH,), jnp.int32),
            pltpu.VMEM((_CH,), jnp.int32),
            pltpu.VMEM((_CH, LATENT), jnp.float32),
            pltpu.VMEM((_CH, LATENT), jnp.float32),
            pltpu.SemaphoreType.DMA,
            pltpu.SemaphoreType.DMA,
            pltpu.SemaphoreType.DMA,
        ],
    )
    def scatter_k(vals, ridx, out, shared,
                  i0, i1, l0, l1, v0, v1,
                  vsem, isem, asem):
        c = lax.axis_index("c")
        s = lax.axis_index("s")
        base = c * _HALF
        ich = (i0, i1)
        lch = (l0, l1)
        vch = (v0, v1)

        def zrow(r, _):
            for k in range(LATENT // 16):
                v0[r, pl.ds(k * 16, 16)] = jnp.zeros((16,), jnp.float32)
            return 0
        lax.fori_loop(0, _ZROWS, zrow, 0)
        for t in range(_ZCH):
            pltpu.sync_copy(
                v0.at[pl.ds(0, _ZROWS)],
                shared.at[pl.ds((s * _ZCH + t) * _ZROWS, _ZROWS)])
        plsc.subcore_barrier()

        def group(g, _):
            vc, ic = [], []
            for b in range(_SNBUF):
                j = g * _SNBUF + b
                vc.append(pltpu.async_copy(vals.at[s, j], vch[b], vsem))
                ic.append(pltpu.async_copy(ridx.at[s, j], ich[b], isem))
            adds = []
            for b in range(_SNBUF):
                ic[b].wait()
                for k in range(_CH // 16):
                    iv = ich[b][pl.ds(k * 16, 16)]
                    loc = iv - base
                    ok = (loc >= 0) & (loc < _HALF)
                    lch[b][pl.ds(k * 16, 16)] = jnp.where(ok, loc, _HALF)
                vc[b].wait()
                adds.append(pltpu.async_copy(
                    vch[b], shared.at[lch[b]], asem, add=True))
            for b in range(_SNBUF):
                adds[b].wait()
            return 0

        lax.fori_loop(0, _SCH // _SNBUF, group, 0)
        plsc.subcore_barrier()

        @pl.when(s < _NS - 1)
        def _copy_full():
            pltpu.sync_copy(shared.at[pl.ds(s * _CPT, _CPT)],
                            out.at[pl.ds(base + s * _CPT, _CPT)])

        @pl.when(s == _NS - 1)
        def _copy_tail():
            pltpu.sync_copy(
                shared.at[pl.ds((_NS - 1) * _CPT, _HALF - (_NS - 1) * _CPT)],
                out.at[pl.ds(base + (_NS - 1) * _CPT,
                             _HALF - (_NS - 1) * _CPT)])

    return scatter_k


_scatter = _make_scatter()


def _relu(x):
    return jnp.maximum(x, 0.0)


def _ln(h, g, b):
    mu = jnp.mean(h, axis=-1, keepdims=True)
    d = h - mu
    var = jnp.mean(d * d, axis=-1, keepdims=True)
    return d * jax.lax.rsqrt(var + 1e-5) * g + b


def _dot(a, b):
    return jax.lax.dot_general(a, b, (((1,), (0,)), ((), ())),
                               preferred_element_type=jnp.float32)


def _bdot(a, b):
    return jax.lax.dot_general(a.astype(jnp.bfloat16), b,
                               (((1,), (0,)), ((), ())),
                               preferred_element_type=jnp.float32)


def _node_enc_body(wp, pwp, nt, w0, b0, w1, b1, w2, b2, g, bb, out, outb):
    vel = wp[...] - pwp[...]
    blk = vel.shape[0]
    oh = (jax.lax.broadcasted_iota(jnp.int32, (blk, NODE_TYPE_SIZE), 1)
          == nt[...]).astype(jnp.float32)
    x = jnp.concatenate([vel, oh], axis=1)
    h = _relu(_dot(x, w0[...]) + b0[...])
    h = _relu(_dot(h, w1[...]) + b1[...])
    o = _dot(h, w2[...]) + b2[...]
    o = _ln(o, g[...], bb[...])
    out[...] = o
    outb[...] = o.astype(jnp.bfloat16)


def _node_encoder(wp, pwp, nt, w0, b0, w1, b1, w2, b2, g, bb):
    grid = N_NODES // _NBLK
    full = lambda s: pl.BlockSpec(s, lambda i: (0, 0))
    oblk = pl.BlockSpec((_NBLK, LATENT), lambda i: (i, 0))
    return pl.pallas_call(
        _node_enc_body,
        grid=(grid,),
        in_specs=[
            pl.BlockSpec((_NBLK, 3), lambda i: (i, 0)),
            pl.BlockSpec((_NBLK, 3), lambda i: (i, 0)),
            pl.BlockSpec((_NBLK, 1), lambda i: (i, 0)),
            full(w0.shape), full(b0.shape), full(w1.shape), full(b1.shape),
            full(w2.shape), full(b2.shape), full(g.shape), full(bb.shape),
        ],
        out_specs=[oblk, oblk],
        out_shape=[jax.ShapeDtypeStruct((N_NODES, LATENT), jnp.float32),
                   jax.ShapeDtypeStruct((N_NODES, LATENT), jnp.bfloat16)],
    )(wp, pwp, nt, w0, b0, w1, b1, w2, b2, g, bb)


def _edge_enc_body(ps, pr, w0, b0, w1, b1, w2, b2, g, bb, out):
    rel = ps[...] - pr[...]
    rw = rel[:, 0:3]
    rm = rel[:, 3:5]
    nw = jnp.sqrt(jnp.sum(rw * rw, axis=1, keepdims=True))
    nm = jnp.sqrt(jnp.sum(rm * rm, axis=1, keepdims=True))
    x = jnp.concatenate([rw, nw, rm, nm], axis=1)
    h = _relu(_dot(x, w0[...]) + b0[...])
    h = _relu(_bdot(h, w1[...]) + b1[...])
    o = _bdot(h, w2[...]) + b2[...]
    out[...] = _ln(o, g[...], bb[...])


def _edge_encoder(ps, pr, w0, b0, w1, b1, w2, b2, g, bb):
    grid = _EPAD // _EBLK
    full = lambda s: pl.BlockSpec(s, lambda i: (0, 0))
    return pl.pallas_call(
        _edge_enc_body,
        grid=(grid,),
        in_specs=[
            pl.BlockSpec((_EBLK, 8), lambda i: (i, 0)),
            pl.BlockSpec((_EBLK, 8), lambda i: (i, 0)),
            full(w0.shape), full(b0.shape), full(w1.shape), full(b1.shape),
            full(w2.shape), full(b2.shape), full(g.shape), full(bb.shape),
        ],
        out_specs=pl.BlockSpec((_EBLK, LATENT), lambda i: (i, 0)),
        out_shape=jax.ShapeDtypeStruct((_EPAD, LATENT), jnp.float32),
    )(ps, pr, w0, b0, w1, b1, w2, b2, g, bb)


def _proc_edge_body(e, s, r, w0a, w0b, w0c, b0, w1, b1, w2, b2, g, bb, out):
    ev = e[...]
    h = _relu(_bdot(ev, w0a[...]) + _bdot(s[...], w0b[...])
              + _bdot(r[...], w0c[...]) + b0[...])
    h = _relu(_bdot(h, w1[...]) + b1[...])
    o = _bdot(h, w2[...]) + b2[...]
    out[...] = ev + _ln(o, g[...], bb[...])


def _proc_edges(e, s, r, w0a, w0b, w0c, b0, w1, b1, w2, b2, g, bb):
    grid = _EPAD // _EBLK
    full = lambda sh: pl.BlockSpec(sh, lambda i: (0, 0))
    blk = pl.BlockSpec((_EBLK, LATENT), lambda i: (i, 0))
    return pl.pallas_call(
        _proc_edge_body,
        grid=(grid,),
        in_specs=[blk, blk, blk,
                  full(w0a.shape), full(w0b.shape), full(w0c.shape),
                  full(b0.shape), full(w1.shape), full(b1.shape),
                  full(w2.shape), full(b2.shape), full(g.shape),
                  full(bb.shape)],
        out_specs=blk,
        out_shape=jax.ShapeDtypeStruct((_EPAD, LATENT), jnp.float32),
    )(e, s, r, w0a, w0b, w0c, b0, w1, b1, w2, b2, g, bb)


def _proc_node_body(n, a, w0a, w0b, b0, w1, b1, w2, b2, g, bb, out, outb):
    nv = n[...]
    h = _relu(_bdot(nv, w0a[...]) + _bdot(a[...], w0b[...]) + b0[...])
    h = _relu(_bdot(h, w1[...]) + b1[...])
    o = _bdot(h, w2[...]) + b2[...]
    o = nv + _ln(o, g[...], bb[...])
    out[...] = o
    outb[...] = o.astype(jnp.bfloat16)


def _proc_nodes(n, a, w0a, w0b, b0, w1, b1, w2, b2, g, bb):
    grid = N_NODES // _NBLK
    full = lambda sh: pl.BlockSpec(sh, lambda i: (0, 0))
    blk = pl.BlockSpec((_NBLK, LATENT), lambda i: (i, 0))
    return pl.pallas_call(
        _proc_node_body,
        grid=(grid,),
        in_specs=[blk, blk,
                  full(w0a.shape), full(w0b.shape), full(b0.shape),
                  full(w1.shape), full(b1.shape), full(w2.shape),
                  full(b2.shape), full(g.shape), full(bb.shape)],
        out_specs=[blk, blk],
        out_shape=[jax.ShapeDtypeStruct((N_NODES, LATENT), jnp.float32),
                   jax.ShapeDtypeStruct((N_NODES, LATENT), jnp.bfloat16)],
    )(n, a, w0a, w0b, b0, w1, b1, w2, b2, g, bb)


def _decoder_body(n, wp, pwp, w0, b0, w1, b1, w2, b2, out):
    h = _relu(_dot(n[...], w0[...]) + b0[...])
    h = _relu(_dot(h, w1[...]) + b1[...])
    o = _dot(h, w2[...]) + b2[...]
    out[...] = 2.0 * wp[...] + o - pwp[...]


def _decoder(n, wp, pwp, w0, b0, w1, b1, w2, b2):
    grid = N_NODES // _NBLK
    full = lambda sh: pl.BlockSpec(sh, lambda i: (0, 0))
    return pl.pallas_call(
        _decoder_body,
        grid=(grid,),
        in_specs=[pl.BlockSpec((_NBLK, LATENT), lambda i: (i, 0)),
                  pl.BlockSpec((_NBLK, 3), lambda i: (i, 0)),
                  pl.BlockSpec((_NBLK, 3), lambda i: (i, 0)),
                  full(w0.shape), full(b0.shape), full(w1.shape),
                  full(b1.shape), full(w2.shape), full(b2.shape)],
        out_specs=pl.BlockSpec((_NBLK, 3), lambda i: (i, 0)),
        out_shape=jax.ShapeDtypeStruct((N_NODES, 3), jnp.float32),
    )(n, wp, pwp, w0, b0, w1, b1, w2, b2)


def _row(v):
    return v.reshape(1, -1)


def _fold_first_layer(p, mean, std):
    w0 = p['w0'] / std[:, None]
    b0 = p['b0'] - (mean / std) @ p['w0']
    return w0, b0


def kernel(world_pos, prev_world_pos, mesh_pos, node_type, senders, receivers,
           params):
    p = params
    senders = senders.astype(jnp.int32)
    receivers = receivers.astype(jnp.int32)

    npad = _EPAD - N_EDGES
    spad = jnp.concatenate(
        [senders, jnp.zeros((npad,), jnp.int32)]).reshape(_NW, _WCH, _CH)
    rpad = jnp.concatenate(
        [receivers, jnp.zeros((npad,), jnp.int32)]).reshape(_NW, _WCH, _CH)
    rscat = jnp.concatenate(
        [receivers, jnp.full((npad,), N_NODES, jnp.int32)]
    ).reshape(_NS, _SCH, _CH)

    ne = p['node_enc']
    nw0, nb0 = _fold_first_layer(ne, p['node_mean'], p['node_std'])
    nodes, nodes_b = _node_encoder(
        world_pos, prev_world_pos, node_type.astype(jnp.int32),
        nw0, _row(nb0), ne['w1'], _row(ne['b1']), ne['w2'], _row(ne['b2']),
        _row(ne['ln_g']), _row(ne['ln_b']))

    pos = jnp.concatenate(
        [world_pos, mesh_pos, jnp.zeros((N_NODES, 3), jnp.float32)], axis=1)
    ps, pr = _sc_gather_pair(pos, spad, rpad, 8)
    ee = p['edge_enc']
    ew0, eb0 = _fold_first_layer(ee, p['edge_mean'], p['edge_std'])
    bf = lambda w: w.astype(jnp.bfloat16)
    edges = _edge_encoder(
        ps, pr, ew0, _row(eb0), bf(ee['w1']), _row(ee['b1']), bf(ee['w2']),
        _row(ee['b2']), _row(ee['ln_g']), _row(ee['ln_b']))

    for i in range(MP_STEPS):
        pe = p['proc_edge_%d' % i]
        pn = p['proc_node_%d' % i]
        sg, rg = _sc_gather_pair(nodes_b, spad, rpad, LATENT)
        w0 = pe['w0']
        edges = _proc_edges(
            edges, sg, rg,
            bf(w0[:LATENT]), bf(w0[LATENT:2 * LATENT]), bf(w0[2 * LATENT:]),
            _row(pe['b0']), bf(pe['w1']), _row(pe['b1']), bf(pe['w2']),
            _row(pe['b2']), _row(pe['ln_g']), _row(pe['ln_b']))
        agg = _scatter(edges.reshape(_NS, _SCH, _CH, LATENT), rscat)
        nw = pn['w0']
        nodes, nodes_b = _proc_nodes(
            nodes, agg, bf(nw[:LATENT]), bf(nw[LATENT:]),
            _row(pn['b0']), bf(pn['w1']), _row(pn['b1']), bf(pn['w2']),
            _row(pn['b2']), _row(pn['ln_g']), _row(pn['ln_b']))

    de = p['decoder']
    dw2 = de['w2'] * p['out_std'][None, :]
    db2 = de['b2'] * p['out_std'] + p['out_mean']
    return _decoder(nodes, world_pos, prev_world_pos,
                    de['w0'], _row(de['b0']), de['w1'], _row(de['b1']),
                    dw2, _row(db2))


# speedup vs baseline: 1.0037x; 1.0037x over previous
import functools

import jax
import jax.numpy as jnp
from jax import lax
from jax.experimental import pallas as pl
from jax.experimental.pallas import tpu as pltpu
from jax.experimental.pallas import tpu_sc as plsc

N_NODES = 50000
N_EDGES = 800000
LATENT = 64
NODE_TYPE_SIZE = 9
MP_STEPS = 2

_NBLK = 2000
_EBLK = 4096

_NC = 2
_NS = 16
_NW = _NC * _NS
_CH = 128
_NBUF = 4
_EPAD = 802816
_WCH = _EPAD // (_NW * _CH)


def _make_gather(depth, dtype):
    mesh = plsc.VectorSubcoreMesh(core_axis_name="c", subcore_axis_name="s")
    oshape = jax.ShapeDtypeStruct((_NW, _WCH, _CH, depth), dtype)

    @functools.partial(
        pl.kernel, mesh=mesh,
        out_type=(oshape, oshape),
        compiler_params=pltpu.CompilerParams(use_tc_tiling_on_sc=False),
        scratch_types=[
            pltpu.VMEM((_WCH, _CH), jnp.int32),
            pltpu.VMEM((_CH, depth), dtype),
            pltpu.VMEM((_CH, depth), dtype),
            pltpu.VMEM((_CH, depth), dtype),
            pltpu.VMEM((_CH, depth), dtype),
            pltpu.SemaphoreType.DMA,
            pltpu.SemaphoreType.DMA,
        ],
    )
    def gather_k(table, idx_a, idx_b, out_a, out_b, idx_v, r0, r1, r2, r3,
                 gsem, ssem):
        wid = lax.axis_index("s") * _NC + lax.axis_index("c")
        rows = (r0, r1, r2, r3)

        def one_pass(idx_hbm, out_hbm):
            pltpu.sync_copy(idx_hbm.at[wid], idx_v)

            def group(g, _):
                cps = []
                for b in range(_NBUF):
                    j = g * _NBUF + b
                    cps.append(pltpu.async_copy(
                        table.at[idx_v.at[j]], rows[b], gsem))
                sts = []
                for b in range(_NBUF):
                    j = g * _NBUF + b
                    cps[b].wait()
                    sts.append(pltpu.async_copy(
                        rows[b], out_hbm.at[wid, j], ssem))
                for b in range(_NBUF):
                    sts[b].wait()
                return 0

            lax.fori_loop(0, _WCH // _NBUF, group, 0)

        one_pass(idx_a, out_a)
        one_pass(idx_b, out_b)

    return gather_k


_gather64 = _make_gather(LATENT, jnp.bfloat16)
_gather8 = _make_gather(8, jnp.float32)


def _sc_gather_pair(table, idx_a, idx_b, depth):
    k = _gather64 if depth == LATENT else _gather8
    oa, ob = k(table, idx_a, idx_b)
    return (oa.reshape(_EPAD, depth), ob.reshape(_EPAD, depth))


_SCH = _EPAD // (_NS * _CH)
_HALF = N_NODES // 2
_TROWS = 25088
_ZROWS = 112
_ZCH = 14
_CPT = 1568
_SNBUF = 2


def _make_scatter():
    mesh = plsc.VectorSubcoreMesh(core_axis_name="c", subcore_axis_name="s")

    @functools.partial(
        pl.kernel, mesh=mesh,
        out_type=jax.ShapeDtypeStruct((N_NODES, LATENT), jnp.float32),
        compiler_params=pltpu.CompilerParams(use_tc_tiling_on_sc=False),
        scratch_types=[
            pltpu.VMEM_SHARED((_TROWS, LATENT), jnp.float32),
            pltpu.VMEM((_CH,), jnp.int32),
            pltpu.VMEM((_CH,), jnp.int32),
            pltpu.VMEM((_CH,), jnp.int32),
            pltpu.VMEM((_CH,), jnp.int32),
            pltpu.VMEM((_CH, LATENT), jnp.float32),
            pltpu.VMEM((_CH, LATENT), jnp.float32),
            pltpu.SemaphoreType.DMA,
            pltpu.SemaphoreType.DMA,
            pltpu.SemaphoreType.DMA,
        ],
    )
    def scatter_k(vals, ridx, out, shared,
                  i0, i1, l0, l1, v0, v1,
                  vsem, isem, asem):
        c = lax.axis_index("c")
        s = lax.axis_index("s")
        base = c * _HALF
        ich = (i0, i1)
        lch = (l0, l1)
        vch = (v0, v1)

        def zrow(r, _):
            for k in range(LATENT // 16):
                v0[r, pl.ds(k * 16, 16)] = jnp.zeros((16,), jnp.float32)
            return 0
        lax.fori_loop(0, _ZROWS, zrow, 0)
        for t in range(_ZCH):
            pltpu.sync_copy(
                v0.at[pl.ds(0, _ZROWS)],
                shared.at[pl.ds((s * _ZCH + t) * _ZROWS, _ZROWS)])
        plsc.subcore_barrier()

        def group(g, _):
            vc, ic = [], []
            for b in range(_SNBUF):
                j = g * _SNBUF + b
                vc.append(pltpu.async_copy(vals.at[s, j], vch[b], vsem))
                ic.append(pltpu.async_copy(ridx.at[s, j], ich[b], isem))
            adds = []
            for b in range(_SNBUF):
                ic[b].wait()
                for k in range(_CH // 16):
                    iv = ich[b][pl.ds(k * 16, 16)]
                    loc = iv - base
                    ok = (loc >= 0) & (loc < _HALF)
                    lch[b][pl.ds(k * 16, 16)] = jnp.where(ok, loc, _HALF)
                vc[b].wait()
                adds.append(pltpu.async_copy(
                    vch[b], shared.at[lch[b]], asem, add=True))
            for b in range(_SNBUF):
                adds[b].wait()
            return 0

        lax.fori_loop(0, _SCH // _SNBUF, group, 0)
        plsc.subcore_barrier()

        @pl.when(s < _NS - 1)
        def _copy_full():
            pltpu.sync_copy(shared.at[pl.ds(s * _CPT, _CPT)],
                            out.at[pl.ds(base + s * _CPT, _CPT)])

        @pl.when(s == _NS - 1)
        def _copy_tail():
            pltpu.sync_copy(
                shared.at[pl.ds((_NS - 1) * _CPT, _HALF - (_NS - 1) * _CPT)],
                out.at[pl.ds(base + (_NS - 1) * _CPT,
                             _HALF - (_NS - 1) * _CPT)])

    return scatter_k


_scatter = _make_scatter()


def _relu(x):
    return jnp.maximum(x, 0.0)


def _ln(h, g, b):
    mu = jnp.mean(h, axis=-1, keepdims=True)
    d = h - mu
    var = jnp.mean(d * d, axis=-1, keepdims=True)
    return d * jax.lax.rsqrt(var + 1e-5) * g + b


def _dot(a, b):
    return jax.lax.dot_general(a, b, (((1,), (0,)), ((), ())),
                               preferred_element_type=jnp.float32)


def _bdot(a, b):
    return jax.lax.dot_general(a.astype(jnp.bfloat16), b,
                               (((1,), (0,)), ((), ())),
                               preferred_element_type=jnp.float32)


def _node_enc_body(wp, pwp, nt, w0, b0, w1, b1, w2, b2, g, bb, out, outb):
    vel = wp[...] - pwp[...]
    blk = vel.shape[0]
    oh = (jax.lax.broadcasted_iota(jnp.int32, (blk, NODE_TYPE_SIZE), 1)
          == nt[...]).astype(jnp.float32)
    x = jnp.concatenate([vel, oh], axis=1)
    h = _relu(_dot(x, w0[...]) + b0[...])
    h = _relu(_dot(h, w1[...]) + b1[...])
    o = _dot(h, w2[...]) + b2[...]
    o = _ln(o, g[...], bb[...])
    out[...] = o
    outb[...] = o.astype(jnp.bfloat16)


def _node_encoder(wp, pwp, nt, w0, b0, w1, b1, w2, b2, g, bb):
    grid = N_NODES // _NBLK
    full = lambda s: pl.BlockSpec(s, lambda i: (0, 0))
    oblk = pl.BlockSpec((_NBLK, LATENT), lambda i: (i, 0))
    return pl.pallas_call(
        _node_enc_body,
        grid=(grid,),
        in_specs=[
            pl.BlockSpec((_NBLK, 3), lambda i: (i, 0)),
            pl.BlockSpec((_NBLK, 3), lambda i: (i, 0)),
            pl.BlockSpec((_NBLK, 1), lambda i: (i, 0)),
            full(w0.shape), full(b0.shape), full(w1.shape), full(b1.shape),
            full(w2.shape), full(b2.shape), full(g.shape), full(bb.shape),
        ],
        out_specs=[oblk, oblk],
        out_shape=[jax.ShapeDtypeStruct((N_NODES, LATENT), jnp.float32),
                   jax.ShapeDtypeStruct((N_NODES, LATENT), jnp.bfloat16)],
    )(wp, pwp, nt, w0, b0, w1, b1, w2, b2, g, bb)


def _edge_enc_body(ps, pr, w0, b0, w1, b1, w2, b2, g, bb, out):
    rel = ps[...] - pr[...]
    rw = rel[:, 0:3]
    rm = rel[:, 3:5]
    nw = jnp.sqrt(jnp.sum(rw * rw, axis=1, keepdims=True))
    nm = jnp.sqrt(jnp.sum(rm * rm, axis=1, keepdims=True))
    x = jnp.concatenate([rw, nw, rm, nm], axis=1)
    h = _relu(_dot(x, w0[...]) + b0[...])
    h = _relu(_dot(h, w1[...]) + b1[...])
    o = _dot(h, w2[...]) + b2[...]
    out[...] = _ln(o, g[...], bb[...])


def _edge_encoder(ps, pr, w0, b0, w1, b1, w2, b2, g, bb):
    grid = _EPAD // _EBLK
    full = lambda s: pl.BlockSpec(s, lambda i: (0, 0))
    return pl.pallas_call(
        _edge_enc_body,
        grid=(grid,),
        in_specs=[
            pl.BlockSpec((_EBLK, 8), lambda i: (i, 0)),
            pl.BlockSpec((_EBLK, 8), lambda i: (i, 0)),
            full(w0.shape), full(b0.shape), full(w1.shape), full(b1.shape),
            full(w2.shape), full(b2.shape), full(g.shape), full(bb.shape),
        ],
        out_specs=pl.BlockSpec((_EBLK, LATENT), lambda i: (i, 0)),
        out_shape=jax.ShapeDtypeStruct((_EPAD, LATENT), jnp.float32),
    )(ps, pr, w0, b0, w1, b1, w2, b2, g, bb)


def _proc_edge_body(e, s, r, w0a, w0b, w0c, b0, w1, b1, w2, b2, g, bb, out):
    ev = e[...]
    sv = s[...].astype(jnp.float32)
    rv = r[...].astype(jnp.float32)
    h = _relu(_dot(ev, w0a[...]) + _dot(sv, w0b[...])
              + _dot(rv, w0c[...]) + b0[...])
    h = _relu(_dot(h, w1[...]) + b1[...])
    o = _dot(h, w2[...]) + b2[...]
    out[...] = ev + _ln(o, g[...], bb[...])


def _proc_edges(e, s, r, w0a, w0b, w0c, b0, w1, b1, w2, b2, g, bb):
    grid = _EPAD // _EBLK
    full = lambda sh: pl.BlockSpec(sh, lambda i: (0, 0))
    blk = pl.BlockSpec((_EBLK, LATENT), lambda i: (i, 0))
    return pl.pallas_call(
        _proc_edge_body,
        grid=(grid,),
        in_specs=[blk, blk, blk,
                  full(w0a.shape), full(w0b.shape), full(w0c.shape),
                  full(b0.shape), full(w1.shape), full(b1.shape),
                  full(w2.shape), full(b2.shape), full(g.shape),
                  full(bb.shape)],
        out_specs=blk,
        out_shape=jax.ShapeDtypeStruct((_EPAD, LATENT), jnp.float32),
    )(e, s, r, w0a, w0b, w0c, b0, w1, b1, w2, b2, g, bb)


def _proc_node_body(n, a, w0a, w0b, b0, w1, b1, w2, b2, g, bb, out, outb):
    nv = n[...]
    h = _relu(_dot(nv, w0a[...]) + _dot(a[...], w0b[...]) + b0[...])
    h = _relu(_dot(h, w1[...]) + b1[...])
    o = _dot(h, w2[...]) + b2[...]
    o = nv + _ln(o, g[...], bb[...])
    out[...] = o
    outb[...] = o.astype(jnp.bfloat16)


def _proc_nodes(n, a, w0a, w0b, b0, w1, b1, w2, b2, g, bb):
    grid = N_NODES // _NBLK
    full = lambda sh: pl.BlockSpec(sh, lambda i: (0, 0))
    blk = pl.BlockSpec((_NBLK, LATENT), lambda i: (i, 0))
    return pl.pallas_call(
        _proc_node_body,
        grid=(grid,),
        in_specs=[blk, blk,
                  full(w0a.shape), full(w0b.shape), full(b0.shape),
                  full(w1.shape), full(b1.shape), full(w2.shape),
                  full(b2.shape), full(g.shape), full(bb.shape)],
        out_specs=[blk, blk],
        out_shape=[jax.ShapeDtypeStruct((N_NODES, LATENT), jnp.float32),
                   jax.ShapeDtypeStruct((N_NODES, LATENT), jnp.bfloat16)],
    )(n, a, w0a, w0b, b0, w1, b1, w2, b2, g, bb)


def _decoder_body(n, wp, pwp, w0, b0, w1, b1, w2, b2, out):
    h = _relu(_dot(n[...], w0[...]) + b0[...])
    h = _relu(_dot(h, w1[...]) + b1[...])
    o = _dot(h, w2[...]) + b2[...]
    out[...] = 2.0 * wp[...] + o - pwp[...]


def _decoder(n, wp, pwp, w0, b0, w1, b1, w2, b2):
    grid = N_NODES // _NBLK
    full = lambda sh: pl.BlockSpec(sh, lambda i: (0, 0))
    return pl.pallas_call(
        _decoder_body,
        grid=(grid,),
        in_specs=[pl.BlockSpec((_NBLK, LATENT), lambda i: (i, 0)),
                  pl.BlockSpec((_NBLK, 3), lambda i: (i, 0)),
                  pl.BlockSpec((_NBLK, 3), lambda i: (i, 0)),
                  full(w0.shape), full(b0.shape), full(w1.shape),
                  full(b1.shape), full(w2.shape), full(b2.shape)],
        out_specs=pl.BlockSpec((_NBLK, 3), lambda i: (i, 0)),
        out_shape=jax.ShapeDtypeStruct((N_NODES, 3), jnp.float32),
    )(n, wp, pwp, w0, b0, w1, b1, w2, b2)


def _row(v):
    return v.reshape(1, -1)


def _fold_first_layer(p, mean, std):
    w0 = p['w0'] / std[:, None]
    b0 = p['b0'] - (mean / std) @ p['w0']
    return w0, b0


def kernel(world_pos, prev_world_pos, mesh_pos, node_type, senders, receivers,
           params):
    p = params
    senders = senders.astype(jnp.int32)
    receivers = receivers.astype(jnp.int32)

    npad = _EPAD - N_EDGES
    spad = jnp.concatenate(
        [senders, jnp.zeros((npad,), jnp.int32)]).reshape(_NW, _WCH, _CH)
    rpad = jnp.concatenate(
        [receivers, jnp.zeros((npad,), jnp.int32)]).reshape(_NW, _WCH, _CH)
    rscat = jnp.concatenate(
        [receivers, jnp.full((npad,), N_NODES, jnp.int32)]
    ).reshape(_NS, _SCH, _CH)

    ne = p['node_enc']
    nw0, nb0 = _fold_first_layer(ne, p['node_mean'], p['node_std'])
    nodes, nodes_b = _node_encoder(
        world_pos, prev_world_pos, node_type.astype(jnp.int32),
        nw0, _row(nb0), ne['w1'], _row(ne['b1']), ne['w2'], _row(ne['b2']),
        _row(ne['ln_g']), _row(ne['ln_b']))

    pos = jnp.concatenate(
        [world_pos, mesh_pos, jnp.zeros((N_NODES, 3), jnp.float32)], axis=1)
    ps, pr = _sc_gather_pair(pos, spad, rpad, 8)
    ee = p['edge_enc']
    ew0, eb0 = _fold_first_layer(ee, p['edge_mean'], p['edge_std'])
    edges = _edge_encoder(
        ps, pr, ew0, _row(eb0), ee['w1'], _row(ee['b1']), ee['w2'],
        _row(ee['b2']), _row(ee['ln_g']), _row(ee['ln_b']))

    for i in range(MP_STEPS):
        pe = p['proc_edge_%d' % i]
        pn = p['proc_node_%d' % i]
        sg, rg = _sc_gather_pair(nodes_b, spad, rpad, LATENT)
        w0 = pe['w0']
        edges = _proc_edges(
            edges, sg, rg,
            w0[:LATENT], w0[LATENT:2 * LATENT], w0[2 * LATENT:],
            _row(pe['b0']), pe['w1'], _row(pe['b1']), pe['w2'],
            _row(pe['b2']), _row(pe['ln_g']), _row(pe['ln_b']))
        agg = _scatter(edges.reshape(_NS, _SCH, _CH, LATENT), rscat)
        nw = pn['w0']
        nodes, nodes_b = _proc_nodes(
            nodes, agg, nw[:LATENT], nw[LATENT:],
            _row(pn['b0']), pn['w1'], _row(pn['b1']), pn['w2'],
            _row(pn['b2']), _row(pn['ln_g']), _row(pn['ln_b']))

    de = p['decoder']
    dw2 = de['w2'] * p['out_std'][None, :]
    db2 = de['b2'] * p['out_std'] + p['out_mean']
    return _decoder(nodes, world_pos, prev_world_pos,
                    de['w0'], _row(de['b0']), de['w1'], _row(de['b1']),
                    dw2, _row(db2))


# speedup vs baseline: 1.0558x; 1.0519x over previous
import functools

import jax
import jax.numpy as jnp
from jax import lax
from jax.experimental import pallas as pl
from jax.experimental.pallas import tpu as pltpu
from jax.experimental.pallas import tpu_sc as plsc

N_NODES = 50000
N_EDGES = 800000
LATENT = 64
NODE_TYPE_SIZE = 9
MP_STEPS = 2

_NBLK = 2000
_EBLK = 4096

_NC = 2
_NS = 16
_NW = _NC * _NS
_CH = 128
_NBUF = 4
_EPAD = 802816
_WCH = _EPAD // (_NW * _CH)


def _make_gather(depth, dtype):
    mesh = plsc.VectorSubcoreMesh(core_axis_name="c", subcore_axis_name="s")
    oshape = jax.ShapeDtypeStruct((_NW, _WCH, _CH, depth), dtype)

    @functools.partial(
        pl.kernel, mesh=mesh,
        out_type=(oshape, oshape),
        compiler_params=pltpu.CompilerParams(use_tc_tiling_on_sc=False),
        scratch_types=[
            pltpu.VMEM((_WCH, _CH), jnp.int32),
            pltpu.VMEM((_CH, depth), dtype),
            pltpu.VMEM((_CH, depth), dtype),
            pltpu.VMEM((_CH, depth), dtype),
            pltpu.VMEM((_CH, depth), dtype),
            pltpu.SemaphoreType.DMA,
            pltpu.SemaphoreType.DMA,
        ],
    )
    def gather_k(table, idx_a, idx_b, out_a, out_b, idx_v, r0, r1, r2, r3,
                 gsem, ssem):
        wid = lax.axis_index("s") * _NC + lax.axis_index("c")
        rows = (r0, r1, r2, r3)

        def one_pass(idx_hbm, out_hbm):
            pltpu.sync_copy(idx_hbm.at[wid], idx_v)

            def group(g, _):
                cps = []
                for b in range(_NBUF):
                    j = g * _NBUF + b
                    cps.append(pltpu.async_copy(
                        table.at[idx_v.at[j]], rows[b], gsem))
                sts = []
                for b in range(_NBUF):
                    j = g * _NBUF + b
                    cps[b].wait()
                    sts.append(pltpu.async_copy(
                        rows[b], out_hbm.at[wid, j], ssem))
                for b in range(_NBUF):
                    sts[b].wait()
                return 0

            lax.fori_loop(0, _WCH // _NBUF, group, 0)

        one_pass(idx_a, out_a)
        one_pass(idx_b, out_b)

    return gather_k


_gather64 = _make_gather(LATENT, jnp.float32)
_gather8 = _make_gather(8, jnp.float32)


def _sc_gather_pair(table, idx_a, idx_b, depth):
    k = _gather64 if depth == LATENT else _gather8
    oa, ob = k(table, idx_a, idx_b)
    return (oa.reshape(_EPAD, depth), ob.reshape(_EPAD, depth))


_SCH = _EPAD // (_NS * _CH)
_HALF = N_NODES // 2
_TROWS = 25088
_ZROWS = 112
_ZCH = 14
_CPT = 1568
_SNBUF = 2


def _make_scatter():
    mesh = plsc.VectorSubcoreMesh(core_axis_name="c", subcore_axis_name="s")

    @functools.partial(
        pl.kernel, mesh=mesh,
        out_type=jax.ShapeDtypeStruct((N_NODES, LATENT), jnp.float32),
        compiler_params=pltpu.CompilerParams(use_tc_tiling_on_sc=False),
        scratch_types=[
            pltpu.VMEM_SHARED((_TROWS, LATENT), jnp.float32),
            pltpu.VMEM((_CH,), jnp.int32),
            pltpu.VMEM((_CH,), jnp.int32),
            pltpu.VMEM((_CH,), jnp.int32),
            pltpu.VMEM((_CH,), jnp.int32),
            pltpu.VMEM((_CH, LATENT), jnp.float32),
            pltpu.VMEM((_CH, LATENT), jnp.float32),
            pltpu.SemaphoreType.DMA,
            pltpu.SemaphoreType.DMA,
            pltpu.SemaphoreType.DMA,
        ],
    )
    def scatter_k(vals, ridx, out, shared,
                  i0, i1, l0, l1, v0, v1,
                  vsem, isem, asem):
        c = lax.axis_index("c")
        s = lax.axis_index("s")
        base = c * _HALF
        ich = (i0, i1)
        lch = (l0, l1)
        vch = (v0, v1)

        def zrow(r, _):
            for k in range(LATENT // 16):
                v0[r, pl.ds(k * 16, 16)] = jnp.zeros((16,), jnp.float32)
            return 0
        lax.fori_loop(0, _ZROWS, zrow, 0)
        for t in range(_ZCH):
            pltpu.sync_copy(
                v0.at[pl.ds(0, _ZROWS)],
                shared.at[pl.ds((s * _ZCH + t) * _ZROWS, _ZROWS)])
        plsc.subcore_barrier()

        def group(g, _):
            vc, ic = [], []
            for b in range(_SNBUF):
                j = g * _SNBUF + b
                vc.append(pltpu.async_copy(vals.at[s, j], vch[b], vsem))
                ic.append(pltpu.async_copy(ridx.at[s, j], ich[b], isem))
            adds = []
            for b in range(_SNBUF):
                ic[b].wait()
                for k in range(_CH // 16):
                    iv = ich[b][pl.ds(k * 16, 16)]
                    loc = iv - base
                    ok = (loc >= 0) & (loc < _HALF)
                    lch[b][pl.ds(k * 16, 16)] = jnp.where(ok, loc, _HALF)
                vc[b].wait()
                adds.append(pltpu.async_copy(
                    vch[b], shared.at[lch[b]], asem, add=True))
            for b in range(_SNBUF):
                adds[b].wait()
            return 0

        lax.fori_loop(0, _SCH // _SNBUF, group, 0)
        plsc.subcore_barrier()

        @pl.when(s < _NS - 1)
        def _copy_full():
            pltpu.sync_copy(shared.at[pl.ds(s * _CPT, _CPT)],
                            out.at[pl.ds(base + s * _CPT, _CPT)])

        @pl.when(s == _NS - 1)
        def _copy_tail():
            pltpu.sync_copy(
                shared.at[pl.ds((_NS - 1) * _CPT, _HALF - (_NS - 1) * _CPT)],
                out.at[pl.ds(base + (_NS - 1) * _CPT,
                             _HALF - (_NS - 1) * _CPT)])

    return scatter_k


_scatter = _make_scatter()


def _relu(x):
    return jnp.maximum(x, 0.0)


def _ln(h, g, b):
    mu = jnp.mean(h, axis=-1, keepdims=True)
    d = h - mu
    var = jnp.mean(d * d, axis=-1, keepdims=True)
    return d * jax.lax.rsqrt(var + 1e-5) * g + b


def _dot(a, b):
    return jax.lax.dot_general(a, b, (((1,), (0,)), ((), ())),
                               preferred_element_type=jnp.float32)


def _bdot(a, b):
    return jax.lax.dot_general(a.astype(jnp.bfloat16), b,
                               (((1,), (0,)), ((), ())),
                               preferred_element_type=jnp.float32)


def _node_enc_body(wp, pwp, nt, w0, b0, w1, b1, w2, b2, g, bb, out):
    vel = wp[...] - pwp[...]
    blk = vel.shape[0]
    oh = (jax.lax.broadcasted_iota(jnp.int32, (blk, NODE_TYPE_SIZE), 1)
          == nt[...]).astype(jnp.float32)
    x = jnp.concatenate([vel, oh], axis=1)
    h = _relu(_dot(x, w0[...]) + b0[...])
    h = _relu(_dot(h, w1[...]) + b1[...])
    o = _dot(h, w2[...]) + b2[...]
    out[...] = _ln(o, g[...], bb[...])


def _node_encoder(wp, pwp, nt, w0, b0, w1, b1, w2, b2, g, bb):
    grid = N_NODES // _NBLK
    full = lambda s: pl.BlockSpec(s, lambda i: (0, 0))
    oblk = pl.BlockSpec((_NBLK, LATENT), lambda i: (i, 0))
    return pl.pallas_call(
        _node_enc_body,
        grid=(grid,),
        in_specs=[
            pl.BlockSpec((_NBLK, 3), lambda i: (i, 0)),
            pl.BlockSpec((_NBLK, 3), lambda i: (i, 0)),
            pl.BlockSpec((_NBLK, 1), lambda i: (i, 0)),
            full(w0.shape), full(b0.shape), full(w1.shape), full(b1.shape),
            full(w2.shape), full(b2.shape), full(g.shape), full(bb.shape),
        ],
        out_specs=oblk,
        out_shape=jax.ShapeDtypeStruct((N_NODES, LATENT), jnp.float32),
    )(wp, pwp, nt, w0, b0, w1, b1, w2, b2, g, bb)


def _edge_enc_body(ps, pr, w0, b0, w1, b1, w2, b2, g, bb, out):
    rel = ps[...] - pr[...]
    rw = rel[:, 0:3]
    rm = rel[:, 3:5]
    nw = jnp.sqrt(jnp.sum(rw * rw, axis=1, keepdims=True))
    nm = jnp.sqrt(jnp.sum(rm * rm, axis=1, keepdims=True))
    x = jnp.concatenate([rw, nw, rm, nm], axis=1)
    h = _relu(_dot(x, w0[...]) + b0[...])
    h = _relu(_dot(h, w1[...]) + b1[...])
    o = _dot(h, w2[...]) + b2[...]
    out[...] = _ln(o, g[...], bb[...])


def _edge_encoder(ps, pr, w0, b0, w1, b1, w2, b2, g, bb):
    grid = _EPAD // _EBLK
    full = lambda s: pl.BlockSpec(s, lambda i: (0, 0))
    return pl.pallas_call(
        _edge_enc_body,
        grid=(grid,),
        in_specs=[
            pl.BlockSpec((_EBLK, 8), lambda i: (i, 0)),
            pl.BlockSpec((_EBLK, 8), lambda i: (i, 0)),
            full(w0.shape), full(b0.shape), full(w1.shape), full(b1.shape),
            full(w2.shape), full(b2.shape), full(g.shape), full(bb.shape),
        ],
        out_specs=pl.BlockSpec((_EBLK, LATENT), lambda i: (i, 0)),
        out_shape=jax.ShapeDtypeStruct((_EPAD, LATENT), jnp.float32),
    )(ps, pr, w0, b0, w1, b1, w2, b2, g, bb)


def _proc_edge_body(e, s, r, w0a, w0b, w0c, b0, w1, b1, w2, b2, g, bb, out):
    ev = e[...]
    h = _relu(_dot(ev, w0a[...]) + _dot(s[...], w0b[...])
              + _dot(r[...], w0c[...]) + b0[...])
    h = _relu(_dot(h, w1[...]) + b1[...])
    o = _dot(h, w2[...]) + b2[...]
    out[...] = ev + _ln(o, g[...], bb[...])


def _proc_edges(e, s, r, w0a, w0b, w0c, b0, w1, b1, w2, b2, g, bb):
    grid = _EPAD // _EBLK
    full = lambda sh: pl.BlockSpec(sh, lambda i: (0, 0))
    blk = pl.BlockSpec((_EBLK, LATENT), lambda i: (i, 0))
    return pl.pallas_call(
        _proc_edge_body,
        grid=(grid,),
        in_specs=[blk, blk, blk,
                  full(w0a.shape), full(w0b.shape), full(w0c.shape),
                  full(b0.shape), full(w1.shape), full(b1.shape),
                  full(w2.shape), full(b2.shape), full(g.shape),
                  full(bb.shape)],
        out_specs=blk,
        out_shape=jax.ShapeDtypeStruct((_EPAD, LATENT), jnp.float32),
    )(e, s, r, w0a, w0b, w0c, b0, w1, b1, w2, b2, g, bb)


def _proc_node_body(n, a, w0a, w0b, b0, w1, b1, w2, b2, g, bb, out):
    nv = n[...]
    h = _relu(_dot(nv, w0a[...]) + _dot(a[...], w0b[...]) + b0[...])
    h = _relu(_dot(h, w1[...]) + b1[...])
    o = _dot(h, w2[...]) + b2[...]
    out[...] = nv + _ln(o, g[...], bb[...])


def _proc_nodes(n, a, w0a, w0b, b0, w1, b1, w2, b2, g, bb):
    grid = N_NODES // _NBLK
    full = lambda sh: pl.BlockSpec(sh, lambda i: (0, 0))
    blk = pl.BlockSpec((_NBLK, LATENT), lambda i: (i, 0))
    return pl.pallas_call(
        _proc_node_body,
        grid=(grid,),
        in_specs=[blk, blk,
                  full(w0a.shape), full(w0b.shape), full(b0.shape),
                  full(w1.shape), full(b1.shape), full(w2.shape),
                  full(b2.shape), full(g.shape), full(bb.shape)],
        out_specs=blk,
        out_shape=jax.ShapeDtypeStruct((N_NODES, LATENT), jnp.float32),
    )(n, a, w0a, w0b, b0, w1, b1, w2, b2, g, bb)


def _decoder_body(n, wp, pwp, w0, b0, w1, b1, w2, b2, out):
    h = _relu(_dot(n[...], w0[...]) + b0[...])
    h = _relu(_dot(h, w1[...]) + b1[...])
    o = _dot(h, w2[...]) + b2[...]
    out[...] = 2.0 * wp[...] + o - pwp[...]


def _decoder(n, wp, pwp, w0, b0, w1, b1, w2, b2):
    grid = N_NODES // _NBLK
    full = lambda sh: pl.BlockSpec(sh, lambda i: (0, 0))
    return pl.pallas_call(
        _decoder_body,
        grid=(grid,),
        in_specs=[pl.BlockSpec((_NBLK, LATENT), lambda i: (i, 0)),
                  pl.BlockSpec((_NBLK, 3), lambda i: (i, 0)),
                  pl.BlockSpec((_NBLK, 3), lambda i: (i, 0)),
                  full(w0.shape), full(b0.shape), full(w1.shape),
                  full(b1.shape), full(w2.shape), full(b2.shape)],
        out_specs=pl.BlockSpec((_NBLK, 3), lambda i: (i, 0)),
        out_shape=jax.ShapeDtypeStruct((N_NODES, 3), jnp.float32),
    )(n, wp, pwp, w0, b0, w1, b1, w2, b2)


def _row(v):
    return v.reshape(1, -1)


def _fold_first_layer(p, mean, std):
    w0 = p['w0'] / std[:, None]
    b0 = p['b0'] - (mean / std) @ p['w0']
    return w0, b0


def kernel(world_pos, prev_world_pos, mesh_pos, node_type, senders, receivers,
           params):
    p = params
    senders = senders.astype(jnp.int32)
    receivers = receivers.astype(jnp.int32)

    npad = _EPAD - N_EDGES
    spad = jnp.concatenate(
        [senders, jnp.zeros((npad,), jnp.int32)]).reshape(_NW, _WCH, _CH)
    rpad = jnp.concatenate(
        [receivers, jnp.zeros((npad,), jnp.int32)]).reshape(_NW, _WCH, _CH)
    rscat = jnp.concatenate(
        [receivers, jnp.full((npad,), N_NODES, jnp.int32)]
    ).reshape(_NS, _SCH, _CH)

    ne = p['node_enc']
    nw0, nb0 = _fold_first_layer(ne, p['node_mean'], p['node_std'])
    nodes = _node_encoder(
        world_pos, prev_world_pos, node_type.astype(jnp.int32),
        nw0, _row(nb0), ne['w1'], _row(ne['b1']), ne['w2'], _row(ne['b2']),
        _row(ne['ln_g']), _row(ne['ln_b']))

    pos = jnp.concatenate(
        [world_pos, mesh_pos, jnp.zeros((N_NODES, 3), jnp.float32)], axis=1)
    ps, pr = _sc_gather_pair(pos, spad, rpad, 8)
    ee = p['edge_enc']
    ew0, eb0 = _fold_first_layer(ee, p['edge_mean'], p['edge_std'])
    edges = _edge_encoder(
        ps, pr, ew0, _row(eb0), ee['w1'], _row(ee['b1']), ee['w2'],
        _row(ee['b2']), _row(ee['ln_g']), _row(ee['ln_b']))

    for i in range(MP_STEPS):
        pe = p['proc_edge_%d' % i]
        pn = p['proc_node_%d' % i]
        sg, rg = _sc_gather_pair(nodes, spad, rpad, LATENT)
        w0 = pe['w0']
        edges = _proc_edges(
            edges, sg, rg,
            w0[:LATENT], w0[LATENT:2 * LATENT], w0[2 * LATENT:],
            _row(pe['b0']), pe['w1'], _row(pe['b1']), pe['w2'],
            _row(pe['b2']), _row(pe['ln_g']), _row(pe['ln_b']))
        agg = _scatter(edges.reshape(_NS, _SCH, _CH, LATENT), rscat)
        nw = pn['w0']
        nodes = _proc_nodes(
            nodes, agg, nw[:LATENT], nw[LATENT:],
            _row(pn['b0']), pn['w1'], _row(pn['b1']), pn['w2'],
            _row(pn['b2']), _row(pn['ln_g']), _row(pn['ln_b']))

    de = p['decoder']
    dw2 = de['w2'] * p['out_std'][None, :]
    db2 = de['b2'] * p['out_std'] + p['out_mean']
    return _decoder(nodes, world_pos, prev_world_pos,
                    de['w0'], _row(de['b0']), de['w1'], _row(de['b1']),
                    dw2, _row(db2))


# speedup vs baseline: 1.1639x; 1.1024x over previous
import functools

import jax
import jax.numpy as jnp
from jax import lax
from jax.experimental import pallas as pl
from jax.experimental.pallas import tpu as pltpu
from jax.experimental.pallas import tpu_sc as plsc

N_NODES = 50000
N_EDGES = 800000
LATENT = 64
NODE_TYPE_SIZE = 9
MP_STEPS = 2

_NBLK = 2000
_EBLK = 4096

_NC = 2
_NS = 16
_NW = _NC * _NS
_CH = 128
_NBUF = 4
_EPAD = 802816
_WCH = _EPAD // (_NW * _CH)


def _make_gather(depth, dtype, wch, nbuf):
    mesh = plsc.VectorSubcoreMesh(core_axis_name="c", subcore_axis_name="s")
    oshape = jax.ShapeDtypeStruct((_NW, wch, _CH, depth), dtype)

    @functools.partial(
        pl.kernel, mesh=mesh,
        out_type=(oshape, oshape),
        compiler_params=pltpu.CompilerParams(use_tc_tiling_on_sc=False),
        scratch_types=[
            pltpu.VMEM((wch, _CH), jnp.int32),
            [pltpu.VMEM((_CH, depth), dtype) for _ in range(nbuf)],
            pltpu.SemaphoreType.DMA,
            pltpu.SemaphoreType.DMA,
        ],
    )
    def gather_k(table, idx_a, idx_b, out_a, out_b, idx_v, rows, gsem, ssem):
        wid = lax.axis_index("s") * _NC + lax.axis_index("c")

        def one_pass(idx_hbm, out_hbm):
            pltpu.sync_copy(idx_hbm.at[wid], idx_v)

            def group(g, _):
                cps = []
                for b in range(nbuf):
                    j = g * nbuf + b
                    cps.append(pltpu.async_copy(
                        table.at[idx_v.at[j]], rows[b], gsem))
                sts = []
                for b in range(nbuf):
                    j = g * nbuf + b
                    cps[b].wait()
                    sts.append(pltpu.async_copy(
                        rows[b], out_hbm.at[wid, j], ssem))
                for b in range(nbuf):
                    sts[b].wait()
                return 0

            lax.fori_loop(0, wch // nbuf, group, 0)

        one_pass(idx_a, out_a)
        one_pass(idx_b, out_b)

    return gather_k


_EHALF = _EPAD // 2
_WCHH = _EHALF // (_NW * _CH)
_gather64 = _make_gather(LATENT, jnp.float32, _WCHH, 7)
_gather8 = _make_gather(8, jnp.float32, _WCHH, 7)


def _sc_gather_pair(table, idx_a, idx_b, depth):
    k = _gather64 if depth == LATENT else _gather8
    oa, ob = k(table, idx_a, idx_b)
    return (oa.reshape(_EHALF, depth), ob.reshape(_EHALF, depth))


_SCH = _EHALF // (_NS * _CH)
_HALF = N_NODES // 2
_TROWS = 25088
_ZROWS = 112
_ZCH = 14
_CPT = 1568
_SNBUF = 2


def _make_scatter():
    mesh = plsc.VectorSubcoreMesh(core_axis_name="c", subcore_axis_name="s")

    @functools.partial(
        pl.kernel, mesh=mesh,
        out_type=jax.ShapeDtypeStruct((N_NODES, LATENT), jnp.float32),
        compiler_params=pltpu.CompilerParams(use_tc_tiling_on_sc=False),
        scratch_types=[
            pltpu.VMEM_SHARED((_TROWS, LATENT), jnp.float32),
            pltpu.VMEM((_CH,), jnp.int32),
            pltpu.VMEM((_CH,), jnp.int32),
            pltpu.VMEM((_CH,), jnp.int32),
            pltpu.VMEM((_CH,), jnp.int32),
            pltpu.VMEM((_CH, LATENT), jnp.float32),
            pltpu.VMEM((_CH, LATENT), jnp.float32),
            pltpu.SemaphoreType.DMA,
            pltpu.SemaphoreType.DMA,
            pltpu.SemaphoreType.DMA,
        ],
    )
    def scatter_k(vals, ridx, out, shared,
                  i0, i1, l0, l1, v0, v1,
                  vsem, isem, asem):
        c = lax.axis_index("c")
        s = lax.axis_index("s")
        base = c * _HALF
        ich = (i0, i1)
        lch = (l0, l1)
        vch = (v0, v1)

        def zrow(r, _):
            for k in range(LATENT // 16):
                v0[r, pl.ds(k * 16, 16)] = jnp.zeros((16,), jnp.float32)
            return 0
        lax.fori_loop(0, _ZROWS, zrow, 0)
        for t in range(_ZCH):
            pltpu.sync_copy(
                v0.at[pl.ds(0, _ZROWS)],
                shared.at[pl.ds((s * _ZCH + t) * _ZROWS, _ZROWS)])
        plsc.subcore_barrier()

        def group(g, _):
            vc, ic = [], []
            for b in range(_SNBUF):
                j = g * _SNBUF + b
                vc.append(pltpu.async_copy(vals.at[s, j], vch[b], vsem))
                ic.append(pltpu.async_copy(ridx.at[s, j], ich[b], isem))
            adds = []
            for b in range(_SNBUF):
                ic[b].wait()
                for k in range(_CH // 16):
                    iv = ich[b][pl.ds(k * 16, 16)]
                    loc = iv - base
                    ok = (loc >= 0) & (loc < _HALF)
                    lch[b][pl.ds(k * 16, 16)] = jnp.where(ok, loc, _HALF)
                vc[b].wait()
                adds.append(pltpu.async_copy(
                    vch[b], shared.at[lch[b]], asem, add=True))
            for b in range(_SNBUF):
                adds[b].wait()
            return 0

        lax.fori_loop(0, _SCH // _SNBUF, group, 0)
        plsc.subcore_barrier()

        @pl.when(s < _NS - 1)
        def _copy_full():
            pltpu.sync_copy(shared.at[pl.ds(s * _CPT, _CPT)],
                            out.at[pl.ds(base + s * _CPT, _CPT)])

        @pl.when(s == _NS - 1)
        def _copy_tail():
            pltpu.sync_copy(
                shared.at[pl.ds((_NS - 1) * _CPT, _HALF - (_NS - 1) * _CPT)],
                out.at[pl.ds(base + (_NS - 1) * _CPT,
                             _HALF - (_NS - 1) * _CPT)])

    return scatter_k


_scatter = _make_scatter()


def _relu(x):
    return jnp.maximum(x, 0.0)


def _ln(h, g, b):
    mu = jnp.mean(h, axis=-1, keepdims=True)
    d = h - mu
    var = jnp.mean(d * d, axis=-1, keepdims=True)
    return d * jax.lax.rsqrt(var + 1e-5) * g + b


def _dot(a, b):
    return jax.lax.dot_general(a, b, (((1,), (0,)), ((), ())),
                               preferred_element_type=jnp.float32)


def _bdot(a, b):
    return jax.lax.dot_general(a.astype(jnp.bfloat16), b,
                               (((1,), (0,)), ((), ())),
                               preferred_element_type=jnp.float32)


def _node_enc_body(wp, pwp, nt, w0, b0, w1, b1, w2, b2, g, bb, out):
    vel = wp[...] - pwp[...]
    blk = vel.shape[0]
    oh = (jax.lax.broadcasted_iota(jnp.int32, (blk, NODE_TYPE_SIZE), 1)
          == nt[...]).astype(jnp.float32)
    x = jnp.concatenate([vel, oh], axis=1)
    h = _relu(_dot(x, w0[...]) + b0[...])
    h = _relu(_dot(h, w1[...]) + b1[...])
    o = _dot(h, w2[...]) + b2[...]
    out[...] = _ln(o, g[...], bb[...])


def _node_encoder(wp, pwp, nt, w0, b0, w1, b1, w2, b2, g, bb):
    grid = N_NODES // _NBLK
    full = lambda s: pl.BlockSpec(s, lambda i: (0, 0))
    oblk = pl.BlockSpec((_NBLK, LATENT), lambda i: (i, 0))
    return pl.pallas_call(
        _node_enc_body,
        grid=(grid,),
        in_specs=[
            pl.BlockSpec((_NBLK, 3), lambda i: (i, 0)),
            pl.BlockSpec((_NBLK, 3), lambda i: (i, 0)),
            pl.BlockSpec((_NBLK, 1), lambda i: (i, 0)),
            full(w0.shape), full(b0.shape), full(w1.shape), full(b1.shape),
            full(w2.shape), full(b2.shape), full(g.shape), full(bb.shape),
        ],
        out_specs=oblk,
        out_shape=jax.ShapeDtypeStruct((N_NODES, LATENT), jnp.float32),
    )(wp, pwp, nt, w0, b0, w1, b1, w2, b2, g, bb)


def _edge_enc_body(ps, pr, w0, b0, w1, b1, w2, b2, g, bb, out):
    rel = ps[...] - pr[...]
    rw = rel[:, 0:3]
    rm = rel[:, 3:5]
    nw = jnp.sqrt(jnp.sum(rw * rw, axis=1, keepdims=True))
    nm = jnp.sqrt(jnp.sum(rm * rm, axis=1, keepdims=True))
    x = jnp.concatenate([rw, nw, rm, nm], axis=1)
    h = _relu(_dot(x, w0[...]) + b0[...])
    h = _relu(_dot(h, w1[...]) + b1[...])
    o = _dot(h, w2[...]) + b2[...]
    out[...] = _ln(o, g[...], bb[...])


def _edge_encoder(ps, pr, w0, b0, w1, b1, w2, b2, g, bb):
    grid = ps.shape[0] // _EBLK
    full = lambda s: pl.BlockSpec(s, lambda i: (0, 0))
    return pl.pallas_call(
        _edge_enc_body,
        grid=(grid,),
        in_specs=[
            pl.BlockSpec((_EBLK, 8), lambda i: (i, 0)),
            pl.BlockSpec((_EBLK, 8), lambda i: (i, 0)),
            full(w0.shape), full(b0.shape), full(w1.shape), full(b1.shape),
            full(w2.shape), full(b2.shape), full(g.shape), full(bb.shape),
        ],
        out_specs=pl.BlockSpec((_EBLK, LATENT), lambda i: (i, 0)),
        out_shape=jax.ShapeDtypeStruct((ps.shape[0], LATENT), jnp.float32),
    )(ps, pr, w0, b0, w1, b1, w2, b2, g, bb)


def _proc_edge_body(e, s, r, w0a, w0b, w0c, b0, w1, b1, w2, b2, g, bb, out):
    ev = e[...]
    h = _relu(_dot(ev, w0a[...]) + _dot(s[...], w0b[...])
              + _dot(r[...], w0c[...]) + b0[...])
    h = _relu(_dot(h, w1[...]) + b1[...])
    o = _dot(h, w2[...]) + b2[...]
    out[...] = ev + _ln(o, g[...], bb[...])


def _proc_edges(e, s, r, w0a, w0b, w0c, b0, w1, b1, w2, b2, g, bb):
    grid = e.shape[0] // _EBLK
    full = lambda sh: pl.BlockSpec(sh, lambda i: (0, 0))
    blk = pl.BlockSpec((_EBLK, LATENT), lambda i: (i, 0))
    return pl.pallas_call(
        _proc_edge_body,
        grid=(grid,),
        in_specs=[blk, blk, blk,
                  full(w0a.shape), full(w0b.shape), full(w0c.shape),
                  full(b0.shape), full(w1.shape), full(b1.shape),
                  full(w2.shape), full(b2.shape), full(g.shape),
                  full(bb.shape)],
        out_specs=blk,
        out_shape=jax.ShapeDtypeStruct((e.shape[0], LATENT), jnp.float32),
    )(e, s, r, w0a, w0b, w0c, b0, w1, b1, w2, b2, g, bb)


def _proc_node_body(n, a, a2, w0a, w0b, b0, w1, b1, w2, b2, g, bb, out):
    nv = n[...]
    av = a[...] + a2[...]
    h = _relu(_dot(nv, w0a[...]) + _dot(av, w0b[...]) + b0[...])
    h = _relu(_dot(h, w1[...]) + b1[...])
    o = _dot(h, w2[...]) + b2[...]
    out[...] = nv + _ln(o, g[...], bb[...])


def _proc_nodes(n, a, a2, w0a, w0b, b0, w1, b1, w2, b2, g, bb):
    grid = N_NODES // _NBLK
    full = lambda sh: pl.BlockSpec(sh, lambda i: (0, 0))
    blk = pl.BlockSpec((_NBLK, LATENT), lambda i: (i, 0))
    return pl.pallas_call(
        _proc_node_body,
        grid=(grid,),
        in_specs=[blk, blk, blk,
                  full(w0a.shape), full(w0b.shape), full(b0.shape),
                  full(w1.shape), full(b1.shape), full(w2.shape),
                  full(b2.shape), full(g.shape), full(bb.shape)],
        out_specs=blk,
        out_shape=jax.ShapeDtypeStruct((N_NODES, LATENT), jnp.float32),
    )(n, a, a2, w0a, w0b, b0, w1, b1, w2, b2, g, bb)


def _decoder_body(n, wp, pwp, w0, b0, w1, b1, w2, b2, out):
    h = _relu(_dot(n[...], w0[...]) + b0[...])
    h = _relu(_dot(h, w1[...]) + b1[...])
    o = _dot(h, w2[...]) + b2[...]
    out[...] = 2.0 * wp[...] + o - pwp[...]


def _decoder(n, wp, pwp, w0, b0, w1, b1, w2, b2):
    grid = N_NODES // _NBLK
    full = lambda sh: pl.BlockSpec(sh, lambda i: (0, 0))
    return pl.pallas_call(
        _decoder_body,
        grid=(grid,),
        in_specs=[pl.BlockSpec((_NBLK, LATENT), lambda i: (i, 0)),
                  pl.BlockSpec((_NBLK, 3), lambda i: (i, 0)),
                  pl.BlockSpec((_NBLK, 3), lambda i: (i, 0)),
                  full(w0.shape), full(b0.shape), full(w1.shape),
                  full(b1.shape), full(w2.shape), full(b2.shape)],
        out_specs=pl.BlockSpec((_NBLK, 3), lambda i: (i, 0)),
        out_shape=jax.ShapeDtypeStruct((N_NODES, 3), jnp.float32),
    )(n, wp, pwp, w0, b0, w1, b1, w2, b2)


def _row(v):
    return v.reshape(1, -1)


def _fold_first_layer(p, mean, std):
    w0 = p['w0'] / std[:, None]
    b0 = p['b0'] - (mean / std) @ p['w0']
    return w0, b0


def kernel(world_pos, prev_world_pos, mesh_pos, node_type, senders, receivers,
           params):
    p = params
    senders = senders.astype(jnp.int32)
    receivers = receivers.astype(jnp.int32)

    npad = _EPAD - N_EDGES
    sflat = jnp.concatenate([senders, jnp.zeros((npad,), jnp.int32)])
    rflat = jnp.concatenate([receivers, jnp.zeros((npad,), jnp.int32)])
    rsflat = jnp.concatenate(
        [receivers, jnp.full((npad,), N_NODES, jnp.int32)])
    spads = [sflat[h * _EHALF:(h + 1) * _EHALF].reshape(_NW, _WCHH, _CH)
             for h in range(2)]
    rpads = [rflat[h * _EHALF:(h + 1) * _EHALF].reshape(_NW, _WCHH, _CH)
             for h in range(2)]
    rscats = [rsflat[h * _EHALF:(h + 1) * _EHALF].reshape(_NS, _SCH, _CH)
              for h in range(2)]

    ne = p['node_enc']
    nw0, nb0 = _fold_first_layer(ne, p['node_mean'], p['node_std'])
    nodes = _node_encoder(
        world_pos, prev_world_pos, node_type.astype(jnp.int32),
        nw0, _row(nb0), ne['w1'], _row(ne['b1']), ne['w2'], _row(ne['b2']),
        _row(ne['ln_g']), _row(ne['ln_b']))

    pos = jnp.concatenate(
        [world_pos, mesh_pos, jnp.zeros((N_NODES, 3), jnp.float32)], axis=1)
    ee = p['edge_enc']
    ew0, eb0 = _fold_first_layer(ee, p['edge_mean'], p['edge_std'])
    edges = []
    for h in range(2):
        ps, pr = _sc_gather_pair(pos, spads[h], rpads[h], 8)
        edges.append(_edge_encoder(
            ps, pr, ew0, _row(eb0), ee['w1'], _row(ee['b1']), ee['w2'],
            _row(ee['b2']), _row(ee['ln_g']), _row(ee['ln_b'])))

    for i in range(MP_STEPS):
        pe = p['proc_edge_%d' % i]
        pn = p['proc_node_%d' % i]
        w0 = pe['w0']
        aggs = []
        for h in range(2):
            sg, rg = _sc_gather_pair(nodes, spads[h], rpads[h], LATENT)
            edges[h] = _proc_edges(
                edges[h], sg, rg,
                w0[:LATENT], w0[LATENT:2 * LATENT], w0[2 * LATENT:],
                _row(pe['b0']), pe['w1'], _row(pe['b1']), pe['w2'],
                _row(pe['b2']), _row(pe['ln_g']), _row(pe['ln_b']))
            aggs.append(_scatter(
                edges[h].reshape(_NS, _SCH, _CH, LATENT), rscats[h]))
        nw = pn['w0']
        nodes = _proc_nodes(
            nodes, aggs[0], aggs[1], nw[:LATENT], nw[LATENT:],
            _row(pn['b0']), pn['w1'], _row(pn['b1']), pn['w2'],
            _row(pn['b2']), _row(pn['ln_g']), _row(pn['ln_b']))

    de = p['decoder']
    dw2 = de['w2'] * p['out_std'][None, :]
    db2 = de['b2'] * p['out_std'] + p['out_mean']
    return _decoder(nodes, world_pos, prev_world_pos,
                    de['w0'], _row(de['b0']), de['w1'], _row(de['b1']),
                    dw2, _row(db2))


# speedup vs baseline: 1.2012x; 1.0320x over previous
import functools

import jax
import jax.numpy as jnp
from jax import lax
from jax.experimental import pallas as pl
from jax.experimental.pallas import tpu as pltpu
from jax.experimental.pallas import tpu_sc as plsc

N_NODES = 50000
N_EDGES = 800000
LATENT = 64
NODE_TYPE_SIZE = 9
MP_STEPS = 2

_NBLK = 5000
_EBLK = 8192

_NC = 2
_NS = 16
_NW = _NC * _NS
_CH = 128
_NBUF = 4
_EPAD = 802816
_WCH = _EPAD // (_NW * _CH)


def _make_gather(depth, dtype, wch, nbuf):
    mesh = plsc.VectorSubcoreMesh(core_axis_name="c", subcore_axis_name="s")
    oshape = jax.ShapeDtypeStruct((_NW, wch, _CH, depth), dtype)

    @functools.partial(
        pl.kernel, mesh=mesh,
        out_type=(oshape, oshape),
        compiler_params=pltpu.CompilerParams(use_tc_tiling_on_sc=False),
        scratch_types=[
            pltpu.VMEM((wch, _CH), jnp.int32),
            [pltpu.VMEM((_CH, depth), dtype) for _ in range(nbuf)],
            pltpu.SemaphoreType.DMA,
            pltpu.SemaphoreType.DMA,
        ],
    )
    def gather_k(table, idx_a, idx_b, out_a, out_b, idx_v, rows, gsem, ssem):
        wid = lax.axis_index("s") * _NC + lax.axis_index("c")

        def one_pass(idx_hbm, out_hbm):
            pltpu.sync_copy(idx_hbm.at[wid], idx_v)

            def group(g, _):
                cps = []
                for b in range(nbuf):
                    j = g * nbuf + b
                    cps.append(pltpu.async_copy(
                        table.at[idx_v.at[j]], rows[b], gsem))
                sts = []
                for b in range(nbuf):
                    j = g * nbuf + b
                    cps[b].wait()
                    sts.append(pltpu.async_copy(
                        rows[b], out_hbm.at[wid, j], ssem))
                for b in range(nbuf):
                    sts[b].wait()
                return 0

            lax.fori_loop(0, wch // nbuf, group, 0)

        one_pass(idx_a, out_a)
        one_pass(idx_b, out_b)

    return gather_k


_EHALF = _EPAD // 2
_WCHH = _EHALF // (_NW * _CH)
_gather64 = _make_gather(LATENT, jnp.float32, _WCHH, 7)
_gather8 = _make_gather(8, jnp.float32, _WCHH, 7)


def _sc_gather_pair(table, idx_a, idx_b, depth):
    k = _gather64 if depth == LATENT else _gather8
    oa, ob = k(table, idx_a, idx_b)
    return (oa.reshape(_EHALF, depth), ob.reshape(_EHALF, depth))


_SCH = _EHALF // (_NS * _CH)
_HALF = N_NODES // 2
_TROWS = 25088
_ZROWS = 112
_ZCH = 14
_CPT = 1568
_SNBUF = 2


def _make_scatter():
    mesh = plsc.VectorSubcoreMesh(core_axis_name="c", subcore_axis_name="s")

    @functools.partial(
        pl.kernel, mesh=mesh,
        out_type=jax.ShapeDtypeStruct((N_NODES, LATENT), jnp.float32),
        compiler_params=pltpu.CompilerParams(use_tc_tiling_on_sc=False),
        scratch_types=[
            pltpu.VMEM_SHARED((_TROWS, LATENT), jnp.float32),
            pltpu.VMEM((_CH,), jnp.int32),
            pltpu.VMEM((_CH,), jnp.int32),
            pltpu.VMEM((_CH,), jnp.int32),
            pltpu.VMEM((_CH,), jnp.int32),
            pltpu.VMEM((_CH, LATENT), jnp.float32),
            pltpu.VMEM((_CH, LATENT), jnp.float32),
            pltpu.SemaphoreType.DMA,
            pltpu.SemaphoreType.DMA,
            pltpu.SemaphoreType.DMA,
        ],
    )
    def scatter_k(vals, ridx, out, shared,
                  i0, i1, l0, l1, v0, v1,
                  vsem, isem, asem):
        c = lax.axis_index("c")
        s = lax.axis_index("s")
        base = c * _HALF
        ich = (i0, i1)
        lch = (l0, l1)
        vch = (v0, v1)

        def zrow(r, _):
            for k in range(LATENT // 16):
                v0[r, pl.ds(k * 16, 16)] = jnp.zeros((16,), jnp.float32)
            return 0
        lax.fori_loop(0, _ZROWS, zrow, 0)
        for t in range(_ZCH):
            pltpu.sync_copy(
                v0.at[pl.ds(0, _ZROWS)],
                shared.at[pl.ds((s * _ZCH + t) * _ZROWS, _ZROWS)])
        plsc.subcore_barrier()

        def group(g, _):
            vc, ic = [], []
            for b in range(_SNBUF):
                j = g * _SNBUF + b
                vc.append(pltpu.async_copy(vals.at[s, j], vch[b], vsem))
                ic.append(pltpu.async_copy(ridx.at[s, j], ich[b], isem))
            adds = []
            for b in range(_SNBUF):
                ic[b].wait()
                for k in range(_CH // 16):
                    iv = ich[b][pl.ds(k * 16, 16)]
                    loc = iv - base
                    ok = (loc >= 0) & (loc < _HALF)
                    lch[b][pl.ds(k * 16, 16)] = jnp.where(ok, loc, _HALF)
                vc[b].wait()
                adds.append(pltpu.async_copy(
                    vch[b], shared.at[lch[b]], asem, add=True))
            for b in range(_SNBUF):
                adds[b].wait()
            return 0

        lax.fori_loop(0, _SCH // _SNBUF, group, 0)
        plsc.subcore_barrier()

        @pl.when(s < _NS - 1)
        def _copy_full():
            pltpu.sync_copy(shared.at[pl.ds(s * _CPT, _CPT)],
                            out.at[pl.ds(base + s * _CPT, _CPT)])

        @pl.when(s == _NS - 1)
        def _copy_tail():
            pltpu.sync_copy(
                shared.at[pl.ds((_NS - 1) * _CPT, _HALF - (_NS - 1) * _CPT)],
                out.at[pl.ds(base + (_NS - 1) * _CPT,
                             _HALF - (_NS - 1) * _CPT)])

    return scatter_k


_scatter = _make_scatter()


def _relu(x):
    return jnp.maximum(x, 0.0)


def _ln(h, g, b):
    mu = jnp.mean(h, axis=-1, keepdims=True)
    d = h - mu
    var = jnp.mean(d * d, axis=-1, keepdims=True)
    return d * jax.lax.rsqrt(var + 1e-5) * g + b


def _dot(a, b):
    return jax.lax.dot_general(a, b, (((1,), (0,)), ((), ())),
                               preferred_element_type=jnp.float32)


def _bdot(a, b):
    return jax.lax.dot_general(a.astype(jnp.bfloat16), b,
                               (((1,), (0,)), ((), ())),
                               preferred_element_type=jnp.float32)


def _node_enc_body(wp, pwp, nt, w0, b0, w1, b1, w2, b2, g, bb, out):
    vel = wp[...] - pwp[...]
    blk = vel.shape[0]
    oh = (jax.lax.broadcasted_iota(jnp.int32, (blk, NODE_TYPE_SIZE), 1)
          == nt[...]).astype(jnp.float32)
    x = jnp.concatenate([vel, oh], axis=1)
    h = _relu(_dot(x, w0[...]) + b0[...])
    h = _relu(_dot(h, w1[...]) + b1[...])
    o = _dot(h, w2[...]) + b2[...]
    out[...] = _ln(o, g[...], bb[...])


def _node_encoder(wp, pwp, nt, w0, b0, w1, b1, w2, b2, g, bb):
    grid = N_NODES // _NBLK
    full = lambda s: pl.BlockSpec(s, lambda i: (0, 0))
    oblk = pl.BlockSpec((_NBLK, LATENT), lambda i: (i, 0))
    return pl.pallas_call(
        _node_enc_body,
        grid=(grid,),
        in_specs=[
            pl.BlockSpec((_NBLK, 3), lambda i: (i, 0)),
            pl.BlockSpec((_NBLK, 3), lambda i: (i, 0)),
            pl.BlockSpec((_NBLK, 1), lambda i: (i, 0)),
            full(w0.shape), full(b0.shape), full(w1.shape), full(b1.shape),
            full(w2.shape), full(b2.shape), full(g.shape), full(bb.shape),
        ],
        out_specs=oblk,
        out_shape=jax.ShapeDtypeStruct((N_NODES, LATENT), jnp.float32),
    )(wp, pwp, nt, w0, b0, w1, b1, w2, b2, g, bb)


def _edge_enc_body(ps, pr, w0, b0, w1, b1, w2, b2, g, bb, out):
    rel = ps[...] - pr[...]
    rw = rel[:, 0:3]
    rm = rel[:, 3:5]
    nw = jnp.sqrt(jnp.sum(rw * rw, axis=1, keepdims=True))
    nm = jnp.sqrt(jnp.sum(rm * rm, axis=1, keepdims=True))
    x = jnp.concatenate([rw, nw, rm, nm], axis=1)
    h = _relu(_dot(x, w0[...]) + b0[...])
    h = _relu(_dot(h, w1[...]) + b1[...])
    o = _dot(h, w2[...]) + b2[...]
    out[...] = _ln(o, g[...], bb[...])


def _edge_encoder(ps, pr, w0, b0, w1, b1, w2, b2, g, bb):
    grid = ps.shape[0] // _EBLK
    full = lambda s: pl.BlockSpec(s, lambda i: (0, 0))
    return pl.pallas_call(
        _edge_enc_body,
        grid=(grid,),
        in_specs=[
            pl.BlockSpec((_EBLK, 8), lambda i: (i, 0)),
            pl.BlockSpec((_EBLK, 8), lambda i: (i, 0)),
            full(w0.shape), full(b0.shape), full(w1.shape), full(b1.shape),
            full(w2.shape), full(b2.shape), full(g.shape), full(bb.shape),
        ],
        out_specs=pl.BlockSpec((_EBLK, LATENT), lambda i: (i, 0)),
        out_shape=jax.ShapeDtypeStruct((ps.shape[0], LATENT), jnp.float32),
    )(ps, pr, w0, b0, w1, b1, w2, b2, g, bb)


def _proc_edge_body(e, s, r, w0a, w0b, w0c, b0, w1, b1, w2, b2, g, bb, out):
    ev = e[...]
    h = _relu(_dot(ev, w0a[...]) + _dot(s[...], w0b[...])
              + _dot(r[...], w0c[...]) + b0[...])
    h = _relu(_dot(h, w1[...]) + b1[...])
    o = _dot(h, w2[...]) + b2[...]
    out[...] = ev + _ln(o, g[...], bb[...])


def _proc_edges(e, s, r, w0a, w0b, w0c, b0, w1, b1, w2, b2, g, bb):
    grid = e.shape[0] // _EBLK
    full = lambda sh: pl.BlockSpec(sh, lambda i: (0, 0))
    blk = pl.BlockSpec((_EBLK, LATENT), lambda i: (i, 0))
    return pl.pallas_call(
        _proc_edge_body,
        grid=(grid,),
        in_specs=[blk, blk, blk,
                  full(w0a.shape), full(w0b.shape), full(w0c.shape),
                  full(b0.shape), full(w1.shape), full(b1.shape),
                  full(w2.shape), full(b2.shape), full(g.shape),
                  full(bb.shape)],
        out_specs=blk,
        out_shape=jax.ShapeDtypeStruct((e.shape[0], LATENT), jnp.float32),
    )(e, s, r, w0a, w0b, w0c, b0, w1, b1, w2, b2, g, bb)


def _proc_node_body(n, a, a2, w0a, w0b, b0, w1, b1, w2, b2, g, bb, out):
    nv = n[...]
    av = a[...] + a2[...]
    h = _relu(_dot(nv, w0a[...]) + _dot(av, w0b[...]) + b0[...])
    h = _relu(_dot(h, w1[...]) + b1[...])
    o = _dot(h, w2[...]) + b2[...]
    out[...] = nv + _ln(o, g[...], bb[...])


def _proc_nodes(n, a, a2, w0a, w0b, b0, w1, b1, w2, b2, g, bb):
    grid = N_NODES // _NBLK
    full = lambda sh: pl.BlockSpec(sh, lambda i: (0, 0))
    blk = pl.BlockSpec((_NBLK, LATENT), lambda i: (i, 0))
    return pl.pallas_call(
        _proc_node_body,
        grid=(grid,),
        in_specs=[blk, blk, blk,
                  full(w0a.shape), full(w0b.shape), full(b0.shape),
                  full(w1.shape), full(b1.shape), full(w2.shape),
                  full(b2.shape), full(g.shape), full(bb.shape)],
        out_specs=blk,
        out_shape=jax.ShapeDtypeStruct((N_NODES, LATENT), jnp.float32),
    )(n, a, a2, w0a, w0b, b0, w1, b1, w2, b2, g, bb)


def _decoder_body(n, wp, pwp, w0, b0, w1, b1, w2, b2, out):
    h = _relu(_dot(n[...], w0[...]) + b0[...])
    h = _relu(_dot(h, w1[...]) + b1[...])
    o = _dot(h, w2[...]) + b2[...]
    out[...] = 2.0 * wp[...] + o - pwp[...]


def _decoder(n, wp, pwp, w0, b0, w1, b1, w2, b2):
    grid = N_NODES // _NBLK
    full = lambda sh: pl.BlockSpec(sh, lambda i: (0, 0))
    return pl.pallas_call(
        _decoder_body,
        grid=(grid,),
        in_specs=[pl.BlockSpec((_NBLK, LATENT), lambda i: (i, 0)),
                  pl.BlockSpec((_NBLK, 3), lambda i: (i, 0)),
                  pl.BlockSpec((_NBLK, 3), lambda i: (i, 0)),
                  full(w0.shape), full(b0.shape), full(w1.shape),
                  full(b1.shape), full(w2.shape), full(b2.shape)],
        out_specs=pl.BlockSpec((_NBLK, 3), lambda i: (i, 0)),
        out_shape=jax.ShapeDtypeStruct((N_NODES, 3), jnp.float32),
    )(n, wp, pwp, w0, b0, w1, b1, w2, b2)


def _row(v):
    return v.reshape(1, -1)


def _fold_first_layer(p, mean, std):
    w0 = p['w0'] / std[:, None]
    b0 = p['b0'] - (mean / std) @ p['w0']
    return w0, b0


def kernel(world_pos, prev_world_pos, mesh_pos, node_type, senders, receivers,
           params):
    p = params
    senders = senders.astype(jnp.int32)
    receivers = receivers.astype(jnp.int32)

    npad = _EPAD - N_EDGES
    sflat = jnp.concatenate([senders, jnp.zeros((npad,), jnp.int32)])
    rflat = jnp.concatenate([receivers, jnp.zeros((npad,), jnp.int32)])
    rsflat = jnp.concatenate(
        [receivers, jnp.full((npad,), N_NODES, jnp.int32)])
    spads = [sflat[h * _EHALF:(h + 1) * _EHALF].reshape(_NW, _WCHH, _CH)
             for h in range(2)]
    rpads = [rflat[h * _EHALF:(h + 1) * _EHALF].reshape(_NW, _WCHH, _CH)
             for h in range(2)]
    rscats = [rsflat[h * _EHALF:(h + 1) * _EHALF].reshape(_NS, _SCH, _CH)
              for h in range(2)]

    ne = p['node_enc']
    nw0, nb0 = _fold_first_layer(ne, p['node_mean'], p['node_std'])
    nodes = _node_encoder(
        world_pos, prev_world_pos, node_type.astype(jnp.int32),
        nw0, _row(nb0), ne['w1'], _row(ne['b1']), ne['w2'], _row(ne['b2']),
        _row(ne['ln_g']), _row(ne['ln_b']))

    pos = jnp.concatenate(
        [world_pos, mesh_pos, jnp.zeros((N_NODES, 3), jnp.float32)], axis=1)
    ee = p['edge_enc']
    ew0, eb0 = _fold_first_layer(ee, p['edge_mean'], p['edge_std'])
    edges = []
    for h in range(2):
        ps, pr = _sc_gather_pair(pos, spads[h], rpads[h], 8)
        edges.append(_edge_encoder(
            ps, pr, ew0, _row(eb0), ee['w1'], _row(ee['b1']), ee['w2'],
            _row(ee['b2']), _row(ee['ln_g']), _row(ee['ln_b'])))

    for i in range(MP_STEPS):
        pe = p['proc_edge_%d' % i]
        pn = p['proc_node_%d' % i]
        w0 = pe['w0']
        aggs = []
        for h in range(2):
            sg, rg = _sc_gather_pair(nodes, spads[h], rpads[h], LATENT)
            edges[h] = _proc_edges(
                edges[h], sg, rg,
                w0[:LATENT], w0[LATENT:2 * LATENT], w0[2 * LATENT:],
                _row(pe['b0']), pe['w1'], _row(pe['b1']), pe['w2'],
                _row(pe['b2']), _row(pe['ln_g']), _row(pe['ln_b']))
            aggs.append(_scatter(
                edges[h].reshape(_NS, _SCH, _CH, LATENT), rscats[h]))
        nw = pn['w0']
        nodes = _proc_nodes(
            nodes, aggs[0], aggs[1], nw[:LATENT], nw[LATENT:],
            _row(pn['b0']), pn['w1'], _row(pn['b1']), pn['w2'],
            _row(pn['b2']), _row(pn['ln_g']), _row(pn['ln_b']))

    de = p['decoder']
    dw2 = de['w2'] * p['out_std'][None, :]
    db2 = de['b2'] * p['out_std'] + p['out_mean']
    return _decoder(nodes, world_pos, prev_world_pos,
                    de['w0'], _row(de['b0']), de['w1'], _row(de['b1']),
                    dw2, _row(db2))


# speedup vs baseline: 1.2017x; 1.0004x over previous
import functools

import jax
import jax.numpy as jnp
from jax import lax
from jax.experimental import pallas as pl
from jax.experimental.pallas import tpu as pltpu
from jax.experimental.pallas import tpu_sc as plsc

N_NODES = 50000
N_EDGES = 800000
LATENT = 64
NODE_TYPE_SIZE = 9
MP_STEPS = 2

_NBLK = 10000
_EBLK = 8192

_NC = 2
_NS = 16
_NW = _NC * _NS
_CH = 128
_NBUF = 4
_EPAD = 802816
_WCH = _EPAD // (_NW * _CH)


def _make_gather(depth, dtype, wch, nbuf):
    mesh = plsc.VectorSubcoreMesh(core_axis_name="c", subcore_axis_name="s")
    oshape = jax.ShapeDtypeStruct((_NW, wch, _CH, depth), dtype)

    @functools.partial(
        pl.kernel, mesh=mesh,
        out_type=(oshape, oshape),
        compiler_params=pltpu.CompilerParams(use_tc_tiling_on_sc=False),
        scratch_types=[
            pltpu.VMEM((wch, _CH), jnp.int32),
            [pltpu.VMEM((_CH, depth), dtype) for _ in range(nbuf)],
            pltpu.SemaphoreType.DMA,
            pltpu.SemaphoreType.DMA,
        ],
    )
    def gather_k(table, idx_a, idx_b, out_a, out_b, idx_v, rows, gsem, ssem):
        wid = lax.axis_index("s") * _NC + lax.axis_index("c")

        def one_pass(idx_hbm, out_hbm):
            pltpu.sync_copy(idx_hbm.at[wid], idx_v)

            def group(g, _):
                cps = []
                for b in range(nbuf):
                    j = g * nbuf + b
                    cps.append(pltpu.async_copy(
                        table.at[idx_v.at[j]], rows[b], gsem))
                sts = []
                for b in range(nbuf):
                    j = g * nbuf + b
                    cps[b].wait()
                    sts.append(pltpu.async_copy(
                        rows[b], out_hbm.at[wid, j], ssem))
                for b in range(nbuf):
                    sts[b].wait()
                return 0

            lax.fori_loop(0, wch // nbuf, group, 0)

        one_pass(idx_a, out_a)
        one_pass(idx_b, out_b)

    return gather_k


_EHALF = _EPAD // 2
_WCHH = _EHALF // (_NW * _CH)
_gather64 = _make_gather(LATENT, jnp.float32, _WCHH, 7)
_gather8 = _make_gather(8, jnp.float32, _WCHH, 7)


def _sc_gather_pair(table, idx_a, idx_b, depth):
    k = _gather64 if depth == LATENT else _gather8
    oa, ob = k(table, idx_a, idx_b)
    return (oa.reshape(_EHALF, depth), ob.reshape(_EHALF, depth))


_SCH = _EHALF // (_NS * _CH)
_HALF = N_NODES // 2
_TROWS = 25088
_ZROWS = 112
_ZCH = 14
_CPT = 1568
_SNBUF = 2


def _make_scatter():
    mesh = plsc.VectorSubcoreMesh(core_axis_name="c", subcore_axis_name="s")

    @functools.partial(
        pl.kernel, mesh=mesh,
        out_type=jax.ShapeDtypeStruct((N_NODES, LATENT), jnp.float32),
        compiler_params=pltpu.CompilerParams(use_tc_tiling_on_sc=False),
        scratch_types=[
            pltpu.VMEM_SHARED((_TROWS, LATENT), jnp.float32),
            pltpu.VMEM((_CH,), jnp.int32),
            pltpu.VMEM((_CH,), jnp.int32),
            pltpu.VMEM((_CH,), jnp.int32),
            pltpu.VMEM((_CH,), jnp.int32),
            pltpu.VMEM((_CH, LATENT), jnp.float32),
            pltpu.VMEM((_CH, LATENT), jnp.float32),
            pltpu.SemaphoreType.DMA,
            pltpu.SemaphoreType.DMA,
            pltpu.SemaphoreType.DMA,
        ],
    )
    def scatter_k(vals, ridx, out, shared,
                  i0, i1, l0, l1, v0, v1,
                  vsem, isem, asem):
        c = lax.axis_index("c")
        s = lax.axis_index("s")
        base = c * _HALF
        ich = (i0, i1)
        lch = (l0, l1)
        vch = (v0, v1)

        def zrow(r, _):
            for k in range(LATENT // 16):
                v0[r, pl.ds(k * 16, 16)] = jnp.zeros((16,), jnp.float32)
            return 0
        lax.fori_loop(0, _ZROWS, zrow, 0)
        for t in range(_ZCH):
            pltpu.sync_copy(
                v0.at[pl.ds(0, _ZROWS)],
                shared.at[pl.ds((s * _ZCH + t) * _ZROWS, _ZROWS)])
        plsc.subcore_barrier()

        def group(g, _):
            vc, ic = [], []
            for b in range(_SNBUF):
                j = g * _SNBUF + b
                vc.append(pltpu.async_copy(vals.at[s, j], vch[b], vsem))
                ic.append(pltpu.async_copy(ridx.at[s, j], ich[b], isem))
            adds = []
            for b in range(_SNBUF):
                ic[b].wait()
                for k in range(_CH // 16):
                    iv = ich[b][pl.ds(k * 16, 16)]
                    loc = iv - base
                    ok = (loc >= 0) & (loc < _HALF)
                    lch[b][pl.ds(k * 16, 16)] = jnp.where(ok, loc, _HALF)
                vc[b].wait()
                adds.append(pltpu.async_copy(
                    vch[b], shared.at[lch[b]], asem, add=True))
            for b in range(_SNBUF):
                adds[b].wait()
            return 0

        lax.fori_loop(0, _SCH // _SNBUF, group, 0)
        plsc.subcore_barrier()

        @pl.when(s < _NS - 1)
        def _copy_full():
            pltpu.sync_copy(shared.at[pl.ds(s * _CPT, _CPT)],
                            out.at[pl.ds(base + s * _CPT, _CPT)])

        @pl.when(s == _NS - 1)
        def _copy_tail():
            pltpu.sync_copy(
                shared.at[pl.ds((_NS - 1) * _CPT, _HALF - (_NS - 1) * _CPT)],
                out.at[pl.ds(base + (_NS - 1) * _CPT,
                             _HALF - (_NS - 1) * _CPT)])

    return scatter_k


_scatter = _make_scatter()


def _relu(x):
    return jnp.maximum(x, 0.0)


def _ln(h, g, b):
    mu = jnp.mean(h, axis=-1, keepdims=True)
    d = h - mu
    var = jnp.mean(d * d, axis=-1, keepdims=True)
    return d * jax.lax.rsqrt(var + 1e-5) * g + b


def _dot(a, b):
    return jax.lax.dot_general(a, b, (((1,), (0,)), ((), ())),
                               preferred_element_type=jnp.float32)


def _bdot(a, b):
    return jax.lax.dot_general(a.astype(jnp.bfloat16), b,
                               (((1,), (0,)), ((), ())),
                               preferred_element_type=jnp.float32)


def _node_enc_body(wp, pwp, nt, w0, b0, w1, b1, w2, b2, g, bb, out):
    vel = wp[...] - pwp[...]
    blk = vel.shape[0]
    oh = (jax.lax.broadcasted_iota(jnp.int32, (blk, NODE_TYPE_SIZE), 1)
          == nt[...]).astype(jnp.float32)
    x = jnp.concatenate([vel, oh], axis=1)
    h = _relu(_dot(x, w0[...]) + b0[...])
    h = _relu(_dot(h, w1[...]) + b1[...])
    o = _dot(h, w2[...]) + b2[...]
    out[...] = _ln(o, g[...], bb[...])


def _node_encoder(wp, pwp, nt, w0, b0, w1, b1, w2, b2, g, bb):
    grid = N_NODES // _NBLK
    full = lambda s: pl.BlockSpec(s, lambda i: (0, 0))
    oblk = pl.BlockSpec((_NBLK, LATENT), lambda i: (i, 0))
    return pl.pallas_call(
        _node_enc_body,
        grid=(grid,),
        in_specs=[
            pl.BlockSpec((_NBLK, 3), lambda i: (i, 0)),
            pl.BlockSpec((_NBLK, 3), lambda i: (i, 0)),
            pl.BlockSpec((_NBLK, 1), lambda i: (i, 0)),
            full(w0.shape), full(b0.shape), full(w1.shape), full(b1.shape),
            full(w2.shape), full(b2.shape), full(g.shape), full(bb.shape),
        ],
        out_specs=oblk,
        out_shape=jax.ShapeDtypeStruct((N_NODES, LATENT), jnp.float32),
    )(wp, pwp, nt, w0, b0, w1, b1, w2, b2, g, bb)


def _edge_enc_body(ps, pr, w0, b0, w1, b1, w2, b2, g, bb, out):
    rel = ps[...] - pr[...]
    rw = rel[:, 0:3]
    rm = rel[:, 3:5]
    nw = jnp.sqrt(jnp.sum(rw * rw, axis=1, keepdims=True))
    nm = jnp.sqrt(jnp.sum(rm * rm, axis=1, keepdims=True))
    x = jnp.concatenate([rw, nw, rm, nm], axis=1)
    h = _relu(_dot(x, w0[...]) + b0[...])
    h = _relu(_dot(h, w1[...]) + b1[...])
    o = _dot(h, w2[...]) + b2[...]
    out[...] = _ln(o, g[...], bb[...])


def _edge_encoder(ps, pr, w0, b0, w1, b1, w2, b2, g, bb):
    grid = ps.shape[0] // _EBLK
    full = lambda s: pl.BlockSpec(s, lambda i: (0, 0))
    return pl.pallas_call(
        _edge_enc_body,
        grid=(grid,),
        in_specs=[
            pl.BlockSpec((_EBLK, 8), lambda i: (i, 0)),
            pl.BlockSpec((_EBLK, 8), lambda i: (i, 0)),
            full(w0.shape), full(b0.shape), full(w1.shape), full(b1.shape),
            full(w2.shape), full(b2.shape), full(g.shape), full(bb.shape),
        ],
        out_specs=pl.BlockSpec((_EBLK, LATENT), lambda i: (i, 0)),
        out_shape=jax.ShapeDtypeStruct((ps.shape[0], LATENT), jnp.float32),
    )(ps, pr, w0, b0, w1, b1, w2, b2, g, bb)


def _proc_edge_body(e, s, r, w0a, w0b, w0c, b0, w1, b1, w2, b2, g, bb, out):
    ev = e[...]
    h = _relu(_dot(ev, w0a[...]) + _dot(s[...], w0b[...])
              + _dot(r[...], w0c[...]) + b0[...])
    h = _relu(_dot(h, w1[...]) + b1[...])
    o = _dot(h, w2[...]) + b2[...]
    out[...] = ev + _ln(o, g[...], bb[...])


def _proc_edges(e, s, r, w0a, w0b, w0c, b0, w1, b1, w2, b2, g, bb):
    grid = e.shape[0] // _EBLK
    full = lambda sh: pl.BlockSpec(sh, lambda i: (0, 0))
    blk = pl.BlockSpec((_EBLK, LATENT), lambda i: (i, 0))
    return pl.pallas_call(
        _proc_edge_body,
        grid=(grid,),
        in_specs=[blk, blk, blk,
                  full(w0a.shape), full(w0b.shape), full(w0c.shape),
                  full(b0.shape), full(w1.shape), full(b1.shape),
                  full(w2.shape), full(b2.shape), full(g.shape),
                  full(bb.shape)],
        out_specs=blk,
        out_shape=jax.ShapeDtypeStruct((e.shape[0], LATENT), jnp.float32),
    )(e, s, r, w0a, w0b, w0c, b0, w1, b1, w2, b2, g, bb)


def _proc_node_body(n, a, a2, w0a, w0b, b0, w1, b1, w2, b2, g, bb, out):
    nv = n[...]
    av = a[...] + a2[...]
    h = _relu(_dot(nv, w0a[...]) + _dot(av, w0b[...]) + b0[...])
    h = _relu(_dot(h, w1[...]) + b1[...])
    o = _dot(h, w2[...]) + b2[...]
    out[...] = nv + _ln(o, g[...], bb[...])


def _proc_nodes(n, a, a2, w0a, w0b, b0, w1, b1, w2, b2, g, bb):
    grid = N_NODES // _NBLK
    full = lambda sh: pl.BlockSpec(sh, lambda i: (0, 0))
    blk = pl.BlockSpec((_NBLK, LATENT), lambda i: (i, 0))
    return pl.pallas_call(
        _proc_node_body,
        grid=(grid,),
        in_specs=[blk, blk, blk,
                  full(w0a.shape), full(w0b.shape), full(b0.shape),
                  full(w1.shape), full(b1.shape), full(w2.shape),
                  full(b2.shape), full(g.shape), full(bb.shape)],
        out_specs=blk,
        out_shape=jax.ShapeDtypeStruct((N_NODES, LATENT), jnp.float32),
    )(n, a, a2, w0a, w0b, b0, w1, b1, w2, b2, g, bb)


def _decoder_body(n, wp, pwp, w0, b0, w1, b1, w2, b2, out):
    h = _relu(_dot(n[...], w0[...]) + b0[...])
    h = _relu(_dot(h, w1[...]) + b1[...])
    o = _dot(h, w2[...]) + b2[...]
    out[...] = 2.0 * wp[...] + o - pwp[...]


def _decoder(n, wp, pwp, w0, b0, w1, b1, w2, b2):
    grid = N_NODES // _NBLK
    full = lambda sh: pl.BlockSpec(sh, lambda i: (0, 0))
    return pl.pallas_call(
        _decoder_body,
        grid=(grid,),
        in_specs=[pl.BlockSpec((_NBLK, LATENT), lambda i: (i, 0)),
                  pl.BlockSpec((_NBLK, 3), lambda i: (i, 0)),
                  pl.BlockSpec((_NBLK, 3), lambda i: (i, 0)),
                  full(w0.shape), full(b0.shape), full(w1.shape),
                  full(b1.shape), full(w2.shape), full(b2.shape)],
        out_specs=pl.BlockSpec((_NBLK, 3), lambda i: (i, 0)),
        out_shape=jax.ShapeDtypeStruct((N_NODES, 3), jnp.float32),
    )(n, wp, pwp, w0, b0, w1, b1, w2, b2)


def _row(v):
    return v.reshape(1, -1)


def _fold_first_layer(p, mean, std):
    w0 = p['w0'] / std[:, None]
    b0 = p['b0'] - (mean / std) @ p['w0']
    return w0, b0


def kernel(world_pos, prev_world_pos, mesh_pos, node_type, senders, receivers,
           params):
    p = params
    senders = senders.astype(jnp.int32)
    receivers = receivers.astype(jnp.int32)

    npad = _EPAD - N_EDGES
    sflat = jnp.concatenate([senders, jnp.zeros((npad,), jnp.int32)])
    rflat = jnp.concatenate([receivers, jnp.zeros((npad,), jnp.int32)])
    rsflat = jnp.concatenate(
        [receivers, jnp.full((npad,), N_NODES, jnp.int32)])
    spads = [sflat[h * _EHALF:(h + 1) * _EHALF].reshape(_NW, _WCHH, _CH)
             for h in range(2)]
    rpads = [rflat[h * _EHALF:(h + 1) * _EHALF].reshape(_NW, _WCHH, _CH)
             for h in range(2)]
    rscats = [rsflat[h * _EHALF:(h + 1) * _EHALF].reshape(_NS, _SCH, _CH)
              for h in range(2)]

    ne = p['node_enc']
    nw0, nb0 = _fold_first_layer(ne, p['node_mean'], p['node_std'])
    nodes = _node_encoder(
        world_pos, prev_world_pos, node_type.astype(jnp.int32),
        nw0, _row(nb0), ne['w1'], _row(ne['b1']), ne['w2'], _row(ne['b2']),
        _row(ne['ln_g']), _row(ne['ln_b']))

    pos = jnp.concatenate(
        [world_pos, mesh_pos, jnp.zeros((N_NODES, 3), jnp.float32)], axis=1)
    ee = p['edge_enc']
    ew0, eb0 = _fold_first_layer(ee, p['edge_mean'], p['edge_std'])
    edges = []
    for h in range(2):
        ps, pr = _sc_gather_pair(pos, spads[h], rpads[h], 8)
        edges.append(_edge_encoder(
            ps, pr, ew0, _row(eb0), ee['w1'], _row(ee['b1']), ee['w2'],
            _row(ee['b2']), _row(ee['ln_g']), _row(ee['ln_b'])))

    for i in range(MP_STEPS):
        pe = p['proc_edge_%d' % i]
        pn = p['proc_node_%d' % i]
        w0 = pe['w0']
        aggs = []
        for h in range(2):
            sg, rg = _sc_gather_pair(nodes, spads[h], rpads[h], LATENT)
            edges[h] = _proc_edges(
                edges[h], sg, rg,
                w0[:LATENT], w0[LATENT:2 * LATENT], w0[2 * LATENT:],
                _row(pe['b0']), pe['w1'], _row(pe['b1']), pe['w2'],
                _row(pe['b2']), _row(pe['ln_g']), _row(pe['ln_b']))
            aggs.append(_scatter(
                edges[h].reshape(_NS, _SCH, _CH, LATENT), rscats[h]))
        nw = pn['w0']
        nodes = _proc_nodes(
            nodes, aggs[0], aggs[1], nw[:LATENT], nw[LATENT:],
            _row(pn['b0']), pn['w1'], _row(pn['b1']), pn['w2'],
            _row(pn['b2']), _row(pn['ln_g']), _row(pn['ln_b']))

    de = p['decoder']
    dw2 = de['w2'] * p['out_std'][None, :]
    db2 = de['b2'] * p['out_std'] + p['out_mean']
    return _decoder(nodes, world_pos, prev_world_pos,
                    de['w0'], _row(de['b0']), de['w1'], _row(de['b1']),
                    dw2, _row(db2))


# speedup vs baseline: 1.2043x; 1.0021x over previous
import functools

import jax
import jax.numpy as jnp
from jax import lax
from jax.experimental import pallas as pl
from jax.experimental.pallas import tpu as pltpu
from jax.experimental.pallas import tpu_sc as plsc

N_NODES = 50000
N_EDGES = 800000
LATENT = 64
NODE_TYPE_SIZE = 9
MP_STEPS = 2

_NBLK = 10000
_EBLK = 8192

_NC = 2
_NS = 16
_NW = _NC * _NS
_CH = 128
_NBUF = 4
_EPAD = 802816
_WCH = _EPAD // (_NW * _CH)


def _make_gather(depth, dtype, wch, nbuf):
    mesh = plsc.VectorSubcoreMesh(core_axis_name="c", subcore_axis_name="s")
    oshape = jax.ShapeDtypeStruct((_NW, wch, _CH, depth), dtype)

    @functools.partial(
        pl.kernel, mesh=mesh,
        out_type=(oshape, oshape),
        compiler_params=pltpu.CompilerParams(use_tc_tiling_on_sc=False),
        scratch_types=[
            pltpu.VMEM((wch, _CH), jnp.int32),
            [pltpu.VMEM((_CH, depth), dtype) for _ in range(nbuf)],
            pltpu.SemaphoreType.DMA,
            pltpu.SemaphoreType.DMA,
        ],
    )
    def gather_k(table, idx_a, idx_b, out_a, out_b, idx_v, rows, gsem, ssem):
        wid = lax.axis_index("s") * _NC + lax.axis_index("c")

        def one_pass(idx_hbm, out_hbm):
            pltpu.sync_copy(idx_hbm.at[wid], idx_v)

            def group(g, _):
                cps = []
                for b in range(nbuf):
                    j = g * nbuf + b
                    cps.append(pltpu.async_copy(
                        table.at[idx_v.at[j]], rows[b], gsem))
                sts = []
                for b in range(nbuf):
                    j = g * nbuf + b
                    cps[b].wait()
                    sts.append(pltpu.async_copy(
                        rows[b], out_hbm.at[wid, j], ssem))
                for b in range(nbuf):
                    sts[b].wait()
                return 0

            lax.fori_loop(0, wch // nbuf, group, 0)

        one_pass(idx_a, out_a)
        one_pass(idx_b, out_b)

    return gather_k


_EHALF = _EPAD // 2
_WCHH = _EHALF // (_NW * _CH)
_gather64 = _make_gather(LATENT, jnp.float32, _WCHH, 14)
_gather8 = _make_gather(8, jnp.float32, _WCHH, 14)


def _sc_gather_pair(table, idx_a, idx_b, depth):
    k = _gather64 if depth == LATENT else _gather8
    oa, ob = k(table, idx_a, idx_b)
    return (oa.reshape(_EHALF, depth), ob.reshape(_EHALF, depth))


_SCH = _EHALF // (_NS * _CH)
_HALF = N_NODES // 2
_TROWS = 25088
_ZROWS = 112
_ZCH = 14
_CPT = 1568
_SNBUF = 2


def _make_scatter():
    mesh = plsc.VectorSubcoreMesh(core_axis_name="c", subcore_axis_name="s")

    @functools.partial(
        pl.kernel, mesh=mesh,
        out_type=jax.ShapeDtypeStruct((N_NODES, LATENT), jnp.float32),
        compiler_params=pltpu.CompilerParams(use_tc_tiling_on_sc=False),
        scratch_types=[
            pltpu.VMEM_SHARED((_TROWS, LATENT), jnp.float32),
            pltpu.VMEM((_CH,), jnp.int32),
            pltpu.VMEM((_CH,), jnp.int32),
            pltpu.VMEM((_CH,), jnp.int32),
            pltpu.VMEM((_CH,), jnp.int32),
            pltpu.VMEM((_CH, LATENT), jnp.float32),
            pltpu.VMEM((_CH, LATENT), jnp.float32),
            pltpu.SemaphoreType.DMA,
            pltpu.SemaphoreType.DMA,
            pltpu.SemaphoreType.DMA,
        ],
    )
    def scatter_k(vals, ridx, out, shared,
                  i0, i1, l0, l1, v0, v1,
                  vsem, isem, asem):
        c = lax.axis_index("c")
        s = lax.axis_index("s")
        base = c * _HALF
        ich = (i0, i1)
        lch = (l0, l1)
        vch = (v0, v1)

        def zrow(r, _):
            for k in range(LATENT // 16):
                v0[r, pl.ds(k * 16, 16)] = jnp.zeros((16,), jnp.float32)
            return 0
        lax.fori_loop(0, _ZROWS, zrow, 0)
        for t in range(_ZCH):
            pltpu.sync_copy(
                v0.at[pl.ds(0, _ZROWS)],
                shared.at[pl.ds((s * _ZCH + t) * _ZROWS, _ZROWS)])
        plsc.subcore_barrier()

        def group(g, _):
            vc, ic = [], []
            for b in range(_SNBUF):
                j = g * _SNBUF + b
                vc.append(pltpu.async_copy(vals.at[s, j], vch[b], vsem))
                ic.append(pltpu.async_copy(ridx.at[s, j], ich[b], isem))
            adds = []
            for b in range(_SNBUF):
                ic[b].wait()
                for k in range(_CH // 16):
                    iv = ich[b][pl.ds(k * 16, 16)]
                    loc = iv - base
                    ok = (loc >= 0) & (loc < _HALF)
                    lch[b][pl.ds(k * 16, 16)] = jnp.where(ok, loc, _HALF)
                vc[b].wait()
                adds.append(pltpu.async_copy(
                    vch[b], shared.at[lch[b]], asem, add=True))
            for b in range(_SNBUF):
                adds[b].wait()
            return 0

        lax.fori_loop(0, _SCH // _SNBUF, group, 0)
        plsc.subcore_barrier()

        @pl.when(s < _NS - 1)
        def _copy_full():
            pltpu.sync_copy(shared.at[pl.ds(s * _CPT, _CPT)],
                            out.at[pl.ds(base + s * _CPT, _CPT)])

        @pl.when(s == _NS - 1)
        def _copy_tail():
            pltpu.sync_copy(
                shared.at[pl.ds((_NS - 1) * _CPT, _HALF - (_NS - 1) * _CPT)],
                out.at[pl.ds(base + (_NS - 1) * _CPT,
                             _HALF - (_NS - 1) * _CPT)])

    return scatter_k


_scatter = _make_scatter()


def _relu(x):
    return jnp.maximum(x, 0.0)


def _ln(h, g, b):
    mu = jnp.mean(h, axis=-1, keepdims=True)
    d = h - mu
    var = jnp.mean(d * d, axis=-1, keepdims=True)
    return d * jax.lax.rsqrt(var + 1e-5) * g + b


def _dot(a, b):
    return jax.lax.dot_general(a, b, (((1,), (0,)), ((), ())),
                               preferred_element_type=jnp.float32)


def _bdot(a, b):
    return jax.lax.dot_general(a.astype(jnp.bfloat16), b,
                               (((1,), (0,)), ((), ())),
                               preferred_element_type=jnp.float32)


def _node_enc_body(wp, pwp, nt, w0, b0, w1, b1, w2, b2, g, bb, out):
    vel = wp[...] - pwp[...]
    blk = vel.shape[0]
    oh = (jax.lax.broadcasted_iota(jnp.int32, (blk, NODE_TYPE_SIZE), 1)
          == nt[...]).astype(jnp.float32)
    x = jnp.concatenate([vel, oh], axis=1)
    h = _relu(_dot(x, w0[...]) + b0[...])
    h = _relu(_dot(h, w1[...]) + b1[...])
    o = _dot(h, w2[...]) + b2[...]
    out[...] = _ln(o, g[...], bb[...])


def _node_encoder(wp, pwp, nt, w0, b0, w1, b1, w2, b2, g, bb):
    grid = N_NODES // _NBLK
    full = lambda s: pl.BlockSpec(s, lambda i: (0, 0))
    oblk = pl.BlockSpec((_NBLK, LATENT), lambda i: (i, 0))
    return pl.pallas_call(
        _node_enc_body,
        grid=(grid,),
        in_specs=[
            pl.BlockSpec((_NBLK, 3), lambda i: (i, 0)),
            pl.BlockSpec((_NBLK, 3), lambda i: (i, 0)),
            pl.BlockSpec((_NBLK, 1), lambda i: (i, 0)),
            full(w0.shape), full(b0.shape), full(w1.shape), full(b1.shape),
            full(w2.shape), full(b2.shape), full(g.shape), full(bb.shape),
        ],
        out_specs=oblk,
        out_shape=jax.ShapeDtypeStruct((N_NODES, LATENT), jnp.float32),
    )(wp, pwp, nt, w0, b0, w1, b1, w2, b2, g, bb)


def _edge_enc_body(ps, pr, w0, b0, w1, b1, w2, b2, g, bb, out):
    rel = ps[...] - pr[...]
    rw = rel[:, 0:3]
    rm = rel[:, 3:5]
    nw = jnp.sqrt(jnp.sum(rw * rw, axis=1, keepdims=True))
    nm = jnp.sqrt(jnp.sum(rm * rm, axis=1, keepdims=True))
    x = jnp.concatenate([rw, nw, rm, nm], axis=1)
    h = _relu(_dot(x, w0[...]) + b0[...])
    h = _relu(_dot(h, w1[...]) + b1[...])
    o = _dot(h, w2[...]) + b2[...]
    out[...] = _ln(o, g[...], bb[...])


def _edge_encoder(ps, pr, w0, b0, w1, b1, w2, b2, g, bb):
    grid = ps.shape[0] // _EBLK
    full = lambda s: pl.BlockSpec(s, lambda i: (0, 0))
    return pl.pallas_call(
        _edge_enc_body,
        grid=(grid,),
        in_specs=[
            pl.BlockSpec((_EBLK, 8), lambda i: (i, 0)),
            pl.BlockSpec((_EBLK, 8), lambda i: (i, 0)),
            full(w0.shape), full(b0.shape), full(w1.shape), full(b1.shape),
            full(w2.shape), full(b2.shape), full(g.shape), full(bb.shape),
        ],
        out_specs=pl.BlockSpec((_EBLK, LATENT), lambda i: (i, 0)),
        out_shape=jax.ShapeDtypeStruct((ps.shape[0], LATENT), jnp.float32),
    )(ps, pr, w0, b0, w1, b1, w2, b2, g, bb)


def _proc_edge_body(e, s, r, w0a, w0b, w0c, b0, w1, b1, w2, b2, g, bb, out):
    ev = e[...]
    h = _relu(_dot(ev, w0a[...]) + _dot(s[...], w0b[...])
              + _dot(r[...], w0c[...]) + b0[...])
    h = _relu(_dot(h, w1[...]) + b1[...])
    o = _dot(h, w2[...]) + b2[...]
    out[...] = ev + _ln(o, g[...], bb[...])


def _proc_edges(e, s, r, w0a, w0b, w0c, b0, w1, b1, w2, b2, g, bb):
    grid = e.shape[0] // _EBLK
    full = lambda sh: pl.BlockSpec(sh, lambda i: (0, 0))
    blk = pl.BlockSpec((_EBLK, LATENT), lambda i: (i, 0))
    return pl.pallas_call(
        _proc_edge_body,
        grid=(grid,),
        in_specs=[blk, blk, blk,
                  full(w0a.shape), full(w0b.shape), full(w0c.shape),
                  full(b0.shape), full(w1.shape), full(b1.shape),
                  full(w2.shape), full(b2.shape), full(g.shape),
                  full(bb.shape)],
        out_specs=blk,
        out_shape=jax.ShapeDtypeStruct((e.shape[0], LATENT), jnp.float32),
    )(e, s, r, w0a, w0b, w0c, b0, w1, b1, w2, b2, g, bb)


def _proc_node_body(n, a, a2, w0a, w0b, b0, w1, b1, w2, b2, g, bb, out):
    nv = n[...]
    av = a[...] + a2[...]
    h = _relu(_dot(nv, w0a[...]) + _dot(av, w0b[...]) + b0[...])
    h = _relu(_dot(h, w1[...]) + b1[...])
    o = _dot(h, w2[...]) + b2[...]
    out[...] = nv + _ln(o, g[...], bb[...])


def _proc_nodes(n, a, a2, w0a, w0b, b0, w1, b1, w2, b2, g, bb):
    grid = N_NODES // _NBLK
    full = lambda sh: pl.BlockSpec(sh, lambda i: (0, 0))
    blk = pl.BlockSpec((_NBLK, LATENT), lambda i: (i, 0))
    return pl.pallas_call(
        _proc_node_body,
        grid=(grid,),
        in_specs=[blk, blk, blk,
                  full(w0a.shape), full(w0b.shape), full(b0.shape),
                  full(w1.shape), full(b1.shape), full(w2.shape),
                  full(b2.shape), full(g.shape), full(bb.shape)],
        out_specs=blk,
        out_shape=jax.ShapeDtypeStruct((N_NODES, LATENT), jnp.float32),
    )(n, a, a2, w0a, w0b, b0, w1, b1, w2, b2, g, bb)


def _decoder_body(n, wp, pwp, w0, b0, w1, b1, w2, b2, out):
    h = _relu(_dot(n[...], w0[...]) + b0[...])
    h = _relu(_dot(h, w1[...]) + b1[...])
    o = _dot(h, w2[...]) + b2[...]
    out[...] = 2.0 * wp[...] + o - pwp[...]


def _decoder(n, wp, pwp, w0, b0, w1, b1, w2, b2):
    grid = N_NODES // _NBLK
    full = lambda sh: pl.BlockSpec(sh, lambda i: (0, 0))
    return pl.pallas_call(
        _decoder_body,
        grid=(grid,),
        in_specs=[pl.BlockSpec((_NBLK, LATENT), lambda i: (i, 0)),
                  pl.BlockSpec((_NBLK, 3), lambda i: (i, 0)),
                  pl.BlockSpec((_NBLK, 3), lambda i: (i, 0)),
                  full(w0.shape), full(b0.shape), full(w1.shape),
                  full(b1.shape), full(w2.shape), full(b2.shape)],
        out_specs=pl.BlockSpec((_NBLK, 3), lambda i: (i, 0)),
        out_shape=jax.ShapeDtypeStruct((N_NODES, 3), jnp.float32),
    )(n, wp, pwp, w0, b0, w1, b1, w2, b2)


def _row(v):
    return v.reshape(1, -1)


def _fold_first_layer(p, mean, std):
    w0 = p['w0'] / std[:, None]
    b0 = p['b0'] - (mean / std) @ p['w0']
    return w0, b0


def kernel(world_pos, prev_world_pos, mesh_pos, node_type, senders, receivers,
           params):
    p = params
    senders = senders.astype(jnp.int32)
    receivers = receivers.astype(jnp.int32)

    npad = _EPAD - N_EDGES
    sflat = jnp.concatenate([senders, jnp.zeros((npad,), jnp.int32)])
    rflat = jnp.concatenate([receivers, jnp.zeros((npad,), jnp.int32)])
    rsflat = jnp.concatenate(
        [receivers, jnp.full((npad,), N_NODES, jnp.int32)])
    spads = [sflat[h * _EHALF:(h + 1) * _EHALF].reshape(_NW, _WCHH, _CH)
             for h in range(2)]
    rpads = [rflat[h * _EHALF:(h + 1) * _EHALF].reshape(_NW, _WCHH, _CH)
             for h in range(2)]
    rscats = [rsflat[h * _EHALF:(h + 1) * _EHALF].reshape(_NS, _SCH, _CH)
              for h in range(2)]

    ne = p['node_enc']
    nw0, nb0 = _fold_first_layer(ne, p['node_mean'], p['node_std'])
    nodes = _node_encoder(
        world_pos, prev_world_pos, node_type.astype(jnp.int32),
        nw0, _row(nb0), ne['w1'], _row(ne['b1']), ne['w2'], _row(ne['b2']),
        _row(ne['ln_g']), _row(ne['ln_b']))

    pos = jnp.concatenate(
        [world_pos, mesh_pos, jnp.zeros((N_NODES, 3), jnp.float32)], axis=1)
    ee = p['edge_enc']
    ew0, eb0 = _fold_first_layer(ee, p['edge_mean'], p['edge_std'])
    edges = []
    for h in range(2):
        ps, pr = _sc_gather_pair(pos, spads[h], rpads[h], 8)
        edges.append(_edge_encoder(
            ps, pr, ew0, _row(eb0), ee['w1'], _row(ee['b1']), ee['w2'],
            _row(ee['b2']), _row(ee['ln_g']), _row(ee['ln_b'])))

    for i in range(MP_STEPS):
        pe = p['proc_edge_%d' % i]
        pn = p['proc_node_%d' % i]
        w0 = pe['w0']
        aggs = []
        for h in range(2):
            sg, rg = _sc_gather_pair(nodes, spads[h], rpads[h], LATENT)
            edges[h] = _proc_edges(
                edges[h], sg, rg,
                w0[:LATENT], w0[LATENT:2 * LATENT], w0[2 * LATENT:],
                _row(pe['b0']), pe['w1'], _row(pe['b1']), pe['w2'],
                _row(pe['b2']), _row(pe['ln_g']), _row(pe['ln_b']))
            aggs.append(_scatter(
                edges[h].reshape(_NS, _SCH, _CH, LATENT), rscats[h]))
        nw = pn['w0']
        nodes = _proc_nodes(
            nodes, aggs[0], aggs[1], nw[:LATENT], nw[LATENT:],
            _row(pn['b0']), pn['w1'], _row(pn['b1']), pn['w2'],
            _row(pn['b2']), _row(pn['ln_g']), _row(pn['ln_b']))

    de = p['decoder']
    dw2 = de['w2'] * p['out_std'][None, :]
    db2 = de['b2'] * p['out_std'] + p['out_mean']
    return _decoder(nodes, world_pos, prev_world_pos,
                    de['w0'], _row(de['b0']), de['w1'], _row(de['b1']),
                    dw2, _row(db2))
